# baseline (speedup 1.0000x reference)
import functools

import jax
import jax.numpy as jnp
from jax import lax
from jax.experimental import pallas as pl
from jax.experimental.pallas import tpu as pltpu
from jax.experimental.pallas import tpu_sc as plsc

N = 50000
E = 800000
F_IN = 8
H = 64
HH = 32
C_OUT = 10
G = 128

NCORE = 2
NSUB = 16
CHUNK = 128
CPB = 8
NBLK = 50
NBLK1 = 25
F_PAD = 16
EPT = CHUNK * CPB * NBLK
E_PAD = EPT * NSUB
EROWS = E_PAD // CHUNK
RPT_E = CPB * NBLK

N_ACC = 50048
RPT = N_ACC // NSUB
ZROWS = 136
WAVE = 4

DEG_PAD = 51200
DPT = DEG_PAD // NSUB

BN = 5000

_MESH = plsc.VectorSubcoreMesh(
    core_axis_name="c", subcore_axis_name="s",
    num_cores=NCORE, num_subcores=NSUB,
)

def _z16():
    return jnp.zeros((16,), jnp.float32)


@functools.partial(
    pl.kernel,
    out_type=jax.ShapeDtypeStruct((NCORE, DEG_PAD), jnp.float32),
    mesh=_MESH,
    compiler_params=pltpu.CompilerParams(use_tc_tiling_on_sc=False),
    scratch_types=[
        pltpu.VMEM_SHARED((DEG_PAD,), jnp.float32),
        pltpu.VMEM((CPB, CHUNK), jnp.int32),
        pltpu.VMEM((CHUNK,), jnp.float32),
        pltpu.VMEM((DPT,), jnp.float32),
    ],
)
def _deg_kernel(dstp, out, acc, dbuf, ones, zbuf):
    c = lax.axis_index("c")
    s = lax.axis_index("s")

    def zfill(i, carry):
        zbuf[pl.ds(i * 16, 16)] = _z16()
        return carry
    lax.fori_loop(0, DPT // 16, zfill, 0)
    pltpu.sync_copy(zbuf, acc.at[pl.ds(s * DPT, DPT)])

    def ofill(i, carry):
        ones[pl.ds(i * 16, 16)] = _z16() + 1.0
        return carry
    lax.fori_loop(0, CHUNK // 16, ofill, 0)
    plsc.subcore_barrier()

    rb0 = (c * NSUB + s) * (NBLK1 * CPB)

    def block(b, carry):
        pltpu.sync_copy(dstp.at[pl.ds(rb0 + b * CPB, CPB)], dbuf)
        for j in range(CPB):
            pltpu.sync_copy(ones, acc.at[dbuf.at[j]], add=True)
        return carry

    lax.fori_loop(0, NBLK1, block, 0)
    plsc.subcore_barrier()
    pltpu.sync_copy(acc.at[pl.ds(s * DPT, DPT)], out.at[c, pl.ds(s * DPT, DPT)])


@functools.partial(
    pl.kernel,
    out_type=jax.ShapeDtypeStruct((NCORE, N_ACC, F_PAD), jnp.float32),
    mesh=_MESH,
    compiler_params=pltpu.CompilerParams(use_tc_tiling_on_sc=False),
    scratch_types=[
        pltpu.VMEM_SHARED((N_ACC, F_PAD), jnp.float32),
        pltpu.VMEM((CPB, CHUNK), jnp.int32),
        pltpu.VMEM((CPB, CHUNK), jnp.int32),
        pltpu.VMEM((CPB * CHUNK, F_PAD), jnp.float32),
        pltpu.VMEM((ZROWS, F_PAD), jnp.float32),
        pltpu.SemaphoreType.DMA,
    ],
)
def _scatter8_kernel(u, srcp, dstp, out, acc, sbuf, dbuf, rows, zbuf, sem):
    c = lax.axis_index("c")
    s = lax.axis_index("s")

    def zrow(r, carry):
        zbuf[r, pl.ds(0, 16)] = _z16()
        return carry
    lax.fori_loop(0, ZROWS, zrow, 0)
    row0 = s * RPT
    for jz in range(RPT // ZROWS):
        pltpu.sync_copy(zbuf, acc.at[pl.ds(row0 + jz * ZROWS, ZROWS)])
    plsc.subcore_barrier()

    rb0 = (c * NSUB + s) * (NBLK1 * CPB)

    def block(b, carry):
        rb = rb0 + b * CPB
        pltpu.sync_copy(srcp.at[pl.ds(rb, CPB)], sbuf)
        pltpu.sync_copy(dstp.at[pl.ds(rb, CPB)], dbuf)
        handles = []
        for j in range(CPB):
            handles.append(pltpu.async_copy(
                u.at[sbuf.at[j]], rows.at[pl.ds(j * CHUNK, CHUNK)], sem))
        for h in handles:
            h.wait()
        for j in range(CPB):
            pltpu.sync_copy(rows.at[pl.ds(j * CHUNK, CHUNK)],
                            acc.at[dbuf.at[j]], add=True)
        return carry

    lax.fori_loop(0, NBLK1, block, 0)
    plsc.subcore_barrier()
    pltpu.sync_copy(acc.at[pl.ds(row0, RPT)], out.at[c, pl.ds(row0, RPT)])


@functools.partial(
    pl.kernel,
    out_type=jax.ShapeDtypeStruct((NCORE, N_ACC, HH), jnp.float32),
    mesh=_MESH,
    compiler_params=pltpu.CompilerParams(use_tc_tiling_on_sc=False),
    scratch_types=[
        pltpu.VMEM_SHARED((N_ACC, HH), jnp.float32),
        pltpu.VMEM((CPB, CHUNK), jnp.int32),
        pltpu.VMEM((CPB, CHUNK), jnp.int32),
        pltpu.VMEM((CPB, CHUNK), jnp.int32),
        pltpu.VMEM((WAVE * CHUNK, HH), jnp.float32),
        pltpu.VMEM((ZROWS, HH), jnp.float32),
        pltpu.SemaphoreType.DMA,
    ],
)
def _scatter_kernel(g2, srcp, dstp, out, acc, sbuf, gbuf, dbuf, rows, zbuf, sem):
    c = lax.axis_index("c")
    s = lax.axis_index("s")

    def zrow(r, carry):
        zbuf[r, pl.ds(0, 16)] = _z16()
        zbuf[r, pl.ds(16, 16)] = _z16()
        return carry
    lax.fori_loop(0, ZROWS, zrow, 0)
    row0 = s * RPT
    for jz in range(RPT // ZROWS):
        pltpu.sync_copy(zbuf, acc.at[pl.ds(row0 + jz * ZROWS, ZROWS)])
    plsc.subcore_barrier()

    rb0 = s * RPT_E

    def block(b, carry):
        rb = rb0 + b * CPB
        pltpu.sync_copy(srcp.at[pl.ds(rb, CPB)], sbuf)
        pltpu.sync_copy(dstp.at[pl.ds(rb, CPB)], dbuf)

        def xf(i, carry2):
            r = i // 8
            k = (i % 8) * 16
            v = sbuf[r, pl.ds(k, 16)]
            gbuf[r, pl.ds(k, 16)] = v + v + c
            return carry2
        lax.fori_loop(0, CPB * 8, xf, 0)

        for w in range(CPB // WAVE):
            handles = []
            for j in range(WAVE):
                handles.append(pltpu.async_copy(
                    g2.at[gbuf.at[w * WAVE + j]],
                    rows.at[pl.ds(j * CHUNK, CHUNK)], sem))
            for h in handles:
                h.wait()
            for j in range(WAVE):
                pltpu.sync_copy(rows.at[pl.ds(j * CHUNK, CHUNK)],
                                acc.at[dbuf.at[w * WAVE + j]], add=True)
        return carry

    lax.fori_loop(0, NBLK, block, 0)
    plsc.subcore_barrier()
    pltpu.sync_copy(acc.at[pl.ds(row0, RPT)], out.at[c, pl.ds(row0, RPT)])


def _pre_body(x_ref, dg_ref, o_ref):
    d = lax.rsqrt(dg_ref[...])
    o_ref[...] = jnp.concatenate(
        [x_ref[...] * d, jnp.zeros((BN, F_PAD - F_IN), jnp.float32)], axis=1)


_pre = pl.pallas_call(
    _pre_body,
    grid=(N // BN,),
    in_specs=[
        pl.BlockSpec((BN, F_IN), lambda i: (i, 0)),
        pl.BlockSpec((BN, 1), lambda i: (i, 0)),
    ],
    out_specs=pl.BlockSpec((BN, F_PAD), lambda i: (i, 0)),
    out_shape=jax.ShapeDtypeStruct((N, F_PAD), jnp.float32),
)


def _mmA_body(t_ref, u_ref, dg_ref, b1_ref, w1_ref, w2_ref, o_ref):
    d = lax.rsqrt(dg_ref[...])
    t = t_ref[0] + t_ref[1] + u_ref[...]
    h = jnp.maximum(
        d * jnp.dot(t[:, :F_IN], w1_ref[...],
                    preferred_element_type=jnp.float32) + b1_ref[...], 0.0)
    o_ref[...] = jnp.dot(h * d, w2_ref[...], preferred_element_type=jnp.float32)


_mmA = pl.pallas_call(
    _mmA_body,
    grid=(N // BN,),
    in_specs=[
        pl.BlockSpec((NCORE, BN, F_PAD), lambda i: (0, i, 0)),
        pl.BlockSpec((BN, F_PAD), lambda i: (i, 0)),
        pl.BlockSpec((BN, 1), lambda i: (i, 0)),
        pl.BlockSpec((1, H), lambda i: (0, 0)),
        pl.BlockSpec((F_IN, H), lambda i: (0, 0)),
        pl.BlockSpec((H, H), lambda i: (0, 0)),
    ],
    out_specs=pl.BlockSpec((BN, H), lambda i: (i, 0)),
    out_shape=jax.ShapeDtypeStruct((N, H), jnp.float32),
)


def _mml_body(s_ref, g_ref, dg_ref, b_ref, w_ref, o_ref):
    d = lax.rsqrt(dg_ref[...])
    sb = jnp.concatenate([s_ref[0], s_ref[1]], axis=1)
    h = jnp.maximum(d * (sb + g_ref[...]) + b_ref[...], 0.0)
    o_ref[...] = jnp.dot(h * d, w_ref[...], preferred_element_type=jnp.float32)


_mml = pl.pallas_call(
    _mml_body,
    grid=(N // BN,),
    in_specs=[
        pl.BlockSpec((NCORE, BN, HH), lambda i: (0, i, 0)),
        pl.BlockSpec((BN, H), lambda i: (i, 0)),
        pl.BlockSpec((BN, 1), lambda i: (i, 0)),
        pl.BlockSpec((1, H), lambda i: (0, 0)),
        pl.BlockSpec((H, H), lambda i: (0, 0)),
    ],
    out_specs=pl.BlockSpec((BN, H), lambda i: (i, 0)),
    out_shape=jax.ShapeDtypeStruct((N, H), jnp.float32),
)


def _pool_body(s_ref, g_ref, dg_ref, b_ref, batch_ref, p_ref):
    i = pl.program_id(0)
    d = lax.rsqrt(dg_ref[...])
    sb = jnp.concatenate([s_ref[0], s_ref[1]], axis=1)
    h = jnp.maximum(d * (sb + g_ref[...]) + b_ref[...], 0.0)
    hh = jnp.concatenate([h, jnp.ones((BN, 1), jnp.float32)], axis=1)
    oh = (batch_ref[...] ==
          lax.broadcasted_iota(jnp.int32, (BN, G), 1)).astype(jnp.float32)
    contrib = lax.dot_general(oh, hh, (((0,), (0,)), ((), ())),
                              preferred_element_type=jnp.float32)

    @pl.when(i == 0)
    def _():
        p_ref[...] = contrib

    @pl.when(i != 0)
    def _():
        p_ref[...] = p_ref[...] + contrib


_pool = pl.pallas_call(
    _pool_body,
    grid=(N // BN,),
    in_specs=[
        pl.BlockSpec((NCORE, BN, HH), lambda i: (0, i, 0)),
        pl.BlockSpec((BN, H), lambda i: (i, 0)),
        pl.BlockSpec((BN, 1), lambda i: (i, 0)),
        pl.BlockSpec((1, H), lambda i: (0, 0)),
        pl.BlockSpec((BN, 1), lambda i: (i, 0)),
    ],
    out_specs=pl.BlockSpec((G, H + 1), lambda i: (0, 0)),
    out_shape=jax.ShapeDtypeStruct((G, H + 1), jnp.float32),
)


def _head_body(p_ref, w_ref, b_ref, o_ref):
    P = p_ref[...]
    cnt = jnp.maximum(P[:, H:H + 1], 1.0)
    pooled = P[:, :H] / cnt
    logits = jnp.dot(pooled, w_ref[...],
                     preferred_element_type=jnp.float32) + b_ref[...]
    m = jnp.max(logits, axis=1, keepdims=True)
    e = jnp.exp(logits - m)
    lse = jnp.log(jnp.sum(e, axis=1, keepdims=True)) + m
    o_ref[...] = logits - lse


_head = pl.pallas_call(
    _head_body,
    out_shape=jax.ShapeDtypeStruct((G, C_OUT), jnp.float32),
)


def kernel(x, edge_index, batch, W1, b1, W2, b2, W3, b3, W4, b4, W5, b5,
           Wout, bout):
    src = edge_index[0]
    dst = edge_index[1]
    pad = E_PAD - E
    srcp = jnp.concatenate(
        [src, jnp.zeros((pad,), jnp.int32)]).reshape(EROWS, CHUNK)
    dstp = jnp.concatenate(
        [dst, jnp.full((pad,), N, jnp.int32)]).reshape(EROWS, CHUNK)

    degp = _deg_kernel(dstp)
    degsum = (1.0 + degp[0, :N] + degp[1, :N]).reshape(N, 1)
    batch2 = batch.reshape(N, 1)

    u1 = _pre(x, degsum)
    T = _scatter8_kernel(u1, srcp, dstp)
    g = _mmA(T, u1, degsum, b1.reshape(1, H), W1, W2)
    for (W, b) in ((W3, b2), (W4, b3), (W5, b4)):
        S = _scatter_kernel(g.reshape(2 * N, HH), srcp, dstp)
        g = _mml(S, g, degsum, b.reshape(1, H), W)
    S = _scatter_kernel(g.reshape(2 * N, HH), srcp, dstp)
    P = _pool(S, g, degsum, b5.reshape(1, H), batch2)
    return _head(P, Wout, bout.reshape(1, C_OUT))


# speedup vs baseline: 1.0011x; 1.0011x over previous
import functools

import jax
import jax.numpy as jnp
from jax import lax
from jax.experimental import pallas as pl
from jax.experimental.pallas import tpu as pltpu
from jax.experimental.pallas import tpu_sc as plsc

N = 50000
E = 800000
F_IN = 8
H = 64
HH = 32
C_OUT = 10
G = 128

NCORE = 2
NSUB = 16
CHUNK = 128
CPB = 8
NBLK = 50
NBLK1 = 25
F_PAD = 16
EPT = CHUNK * CPB * NBLK
E_PAD = EPT * NSUB
EROWS = E_PAD // CHUNK
RPT_E = CPB * NBLK

N_ACC = 50048
RPT = N_ACC // NSUB
ZROWS = 136
WAVE = 4

DEG_PAD = 51200
DPT = DEG_PAD // NSUB

BN = 5000

_MESH = plsc.VectorSubcoreMesh(
    core_axis_name="c", subcore_axis_name="s",
    num_cores=NCORE, num_subcores=NSUB,
)

def _z16():
    return jnp.zeros((16,), jnp.float32)


@functools.partial(
    pl.kernel,
    out_type=jax.ShapeDtypeStruct((NCORE, DEG_PAD), jnp.float32),
    mesh=_MESH,
    compiler_params=pltpu.CompilerParams(use_tc_tiling_on_sc=False),
    scratch_types=[
        pltpu.VMEM_SHARED((DEG_PAD,), jnp.float32),
        pltpu.VMEM((CPB, CHUNK), jnp.int32),
        pltpu.VMEM((CHUNK,), jnp.float32),
        pltpu.VMEM((DPT,), jnp.float32),
    ],
)
def _deg_kernel(dstp, out, acc, dbuf, ones, zbuf):
    c = lax.axis_index("c")
    s = lax.axis_index("s")

    def zfill(i, carry):
        zbuf[pl.ds(i * 16, 16)] = _z16()
        return carry
    lax.fori_loop(0, DPT // 16, zfill, 0)
    pltpu.sync_copy(zbuf, acc.at[pl.ds(s * DPT, DPT)])

    def ofill(i, carry):
        ones[pl.ds(i * 16, 16)] = _z16() + 1.0
        return carry
    lax.fori_loop(0, CHUNK // 16, ofill, 0)
    plsc.subcore_barrier()

    rb0 = (c * NSUB + s) * (NBLK1 * CPB)

    def block(b, carry):
        pltpu.sync_copy(dstp.at[pl.ds(rb0 + b * CPB, CPB)], dbuf)
        for j in range(CPB):
            pltpu.sync_copy(ones, acc.at[dbuf.at[j]], add=True)
        return carry

    lax.fori_loop(0, NBLK1, block, 0)
    plsc.subcore_barrier()
    pltpu.sync_copy(acc.at[pl.ds(s * DPT, DPT)], out.at[c, pl.ds(s * DPT, DPT)])


@functools.partial(
    pl.kernel,
    out_type=jax.ShapeDtypeStruct((NCORE, N_ACC, F_PAD), jnp.float32),
    mesh=_MESH,
    compiler_params=pltpu.CompilerParams(use_tc_tiling_on_sc=False),
    scratch_types=[
        pltpu.VMEM_SHARED((N_ACC, F_PAD), jnp.float32),
        pltpu.VMEM((CPB, CHUNK), jnp.int32),
        pltpu.VMEM((CPB, CHUNK), jnp.int32),
        pltpu.VMEM((CPB * CHUNK, F_PAD), jnp.float32),
        pltpu.VMEM((ZROWS, F_PAD), jnp.float32),
        pltpu.SemaphoreType.DMA,
    ],
)
def _scatter8_kernel(u, srcp, dstp, out, acc, sbuf, dbuf, rows, zbuf, sem):
    c = lax.axis_index("c")
    s = lax.axis_index("s")

    def zrow(r, carry):
        zbuf[r, pl.ds(0, 16)] = _z16()
        return carry
    lax.fori_loop(0, ZROWS, zrow, 0)
    row0 = s * RPT
    for jz in range(RPT // ZROWS):
        pltpu.sync_copy(zbuf, acc.at[pl.ds(row0 + jz * ZROWS, ZROWS)])
    plsc.subcore_barrier()

    rb0 = (c * NSUB + s) * (NBLK1 * CPB)

    def block(b, carry):
        rb = rb0 + b * CPB
        pltpu.sync_copy(srcp.at[pl.ds(rb, CPB)], sbuf)
        pltpu.sync_copy(dstp.at[pl.ds(rb, CPB)], dbuf)
        handles = []
        for j in range(CPB):
            handles.append(pltpu.async_copy(
                u.at[sbuf.at[j]], rows.at[pl.ds(j * CHUNK, CHUNK)], sem))
        for h in handles:
            h.wait()
        for j in range(CPB):
            pltpu.sync_copy(rows.at[pl.ds(j * CHUNK, CHUNK)],
                            acc.at[dbuf.at[j]], add=True)
        return carry

    lax.fori_loop(0, NBLK1, block, 0)
    plsc.subcore_barrier()
    pltpu.sync_copy(acc.at[pl.ds(row0, RPT)], out.at[c, pl.ds(row0, RPT)])


@functools.partial(
    pl.kernel,
    out_type=jax.ShapeDtypeStruct((NCORE, N_ACC, HH), jnp.float32),
    mesh=_MESH,
    compiler_params=pltpu.CompilerParams(use_tc_tiling_on_sc=False),
    scratch_types=[
        pltpu.VMEM_SHARED((N_ACC, HH), jnp.float32),
        pltpu.VMEM((CPB, CHUNK), jnp.int32),
        pltpu.VMEM((CPB, CHUNK), jnp.int32),
        pltpu.VMEM((CPB, CHUNK), jnp.int32),
        pltpu.VMEM((WAVE * CHUNK, HH), jnp.float32),
        pltpu.VMEM((ZROWS, HH), jnp.float32),
        pltpu.SemaphoreType.DMA,
    ],
)
def _scatter_kernel(g2, srcp, dstp, out, acc, sbuf, gbuf, dbuf, rows, zbuf, sem):
    c = lax.axis_index("c")
    s = lax.axis_index("s")

    def zrow(r, carry):
        zbuf[r, pl.ds(0, 16)] = _z16()
        zbuf[r, pl.ds(16, 16)] = _z16()
        return carry
    lax.fori_loop(0, ZROWS, zrow, 0)
    row0 = s * RPT
    for jz in range(RPT // ZROWS):
        pltpu.sync_copy(zbuf, acc.at[pl.ds(row0 + jz * ZROWS, ZROWS)])
    plsc.subcore_barrier()

    rb0 = s * RPT_E

    def block(b, carry):
        rb = rb0 + b * CPB
        pltpu.sync_copy(srcp.at[pl.ds(rb, CPB)], sbuf)
        pltpu.sync_copy(dstp.at[pl.ds(rb, CPB)], dbuf)

        def xf(i, carry2):
            r = i // 8
            k = (i % 8) * 16
            v = sbuf[r, pl.ds(k, 16)]
            gbuf[r, pl.ds(k, 16)] = v + v + c
            return carry2
        lax.fori_loop(0, CPB * 8, xf, 0)

        for w in range(CPB // WAVE):
            handles = []
            for j in range(WAVE):
                handles.append(pltpu.async_copy(
                    g2.at[gbuf.at[w * WAVE + j]],
                    rows.at[pl.ds(j * CHUNK, CHUNK)], sem))
            for h in handles:
                h.wait()
            for j in range(WAVE):
                pltpu.sync_copy(rows.at[pl.ds(j * CHUNK, CHUNK)],
                                acc.at[dbuf.at[w * WAVE + j]], add=True)
        return carry

    lax.fori_loop(0, NBLK, block, 0)
    plsc.subcore_barrier()
    pltpu.sync_copy(acc.at[pl.ds(row0, RPT)], out.at[c, pl.ds(row0, RPT)])


def _pre_body(x_ref, dg_ref, o_ref):
    d = lax.rsqrt(dg_ref[...])
    o_ref[...] = jnp.concatenate(
        [x_ref[...] * d, jnp.zeros((BN, F_PAD - F_IN), jnp.float32)], axis=1)


_pre = pl.pallas_call(
    _pre_body,
    grid=(N // BN,),
    in_specs=[
        pl.BlockSpec((BN, F_IN), lambda i: (i, 0)),
        pl.BlockSpec((BN, 1), lambda i: (i, 0)),
    ],
    out_specs=pl.BlockSpec((BN, F_PAD), lambda i: (i, 0)),
    out_shape=jax.ShapeDtypeStruct((N, F_PAD), jnp.float32),
)


def _mmA_body(t_ref, u_ref, dg_ref, b1_ref, w1_ref, w2_ref, o_ref):
    d = lax.rsqrt(dg_ref[...])
    t = t_ref[0] + t_ref[1] + u_ref[...]
    h = jnp.maximum(
        d * jnp.dot(t[:, :F_IN], w1_ref[...],
                    preferred_element_type=jnp.float32) + b1_ref[...], 0.0)
    o_ref[...] = jnp.dot(h * d, w2_ref[...], preferred_element_type=jnp.float32)


_mmA = pl.pallas_call(
    _mmA_body,
    grid=(N // BN,),
    in_specs=[
        pl.BlockSpec((NCORE, BN, F_PAD), lambda i: (0, i, 0)),
        pl.BlockSpec((BN, F_PAD), lambda i: (i, 0)),
        pl.BlockSpec((BN, 1), lambda i: (i, 0)),
        pl.BlockSpec((1, H), lambda i: (0, 0)),
        pl.BlockSpec((F_IN, H), lambda i: (0, 0)),
        pl.BlockSpec((H, H), lambda i: (0, 0)),
    ],
    out_specs=pl.BlockSpec((BN, H), lambda i: (i, 0)),
    out_shape=jax.ShapeDtypeStruct((N, H), jnp.float32),
)


def _mml_body(s_ref, g_ref, dg_ref, b_ref, w_ref, o_ref):
    d = lax.rsqrt(dg_ref[...])
    sb = jnp.concatenate([s_ref[0], s_ref[1]], axis=1)
    h = jnp.maximum(d * (sb + g_ref[...]) + b_ref[...], 0.0)
    o_ref[...] = jnp.dot(h * d, w_ref[...], preferred_element_type=jnp.float32)


_mml = pl.pallas_call(
    _mml_body,
    grid=(N // BN,),
    in_specs=[
        pl.BlockSpec((NCORE, BN, HH), lambda i: (0, i, 0)),
        pl.BlockSpec((BN, H), lambda i: (i, 0)),
        pl.BlockSpec((BN, 1), lambda i: (i, 0)),
        pl.BlockSpec((1, H), lambda i: (0, 0)),
        pl.BlockSpec((H, H), lambda i: (0, 0)),
    ],
    out_specs=pl.BlockSpec((BN, H), lambda i: (i, 0)),
    out_shape=jax.ShapeDtypeStruct((N, H), jnp.float32),
)


def _pool_body(s_ref, g_ref, dg_ref, b_ref, batch_ref, p_ref):
    i = pl.program_id(0)
    d = lax.rsqrt(dg_ref[...])
    sb = jnp.concatenate([s_ref[0], s_ref[1]], axis=1)
    h = jnp.maximum(d * (sb + g_ref[...]) + b_ref[...], 0.0)
    hh = jnp.concatenate([h, jnp.ones((BN, 1), jnp.float32)], axis=1)
    oh = (batch_ref[...] ==
          lax.broadcasted_iota(jnp.int32, (BN, G), 1)).astype(jnp.float32)
    contrib = lax.dot_general(oh, hh, (((0,), (0,)), ((), ())),
                              preferred_element_type=jnp.float32)

    @pl.when(i == 0)
    def _():
        p_ref[...] = contrib

    @pl.when(i != 0)
    def _():
        p_ref[...] = p_ref[...] + contrib


_pool = pl.pallas_call(
    _pool_body,
    grid=(N // BN,),
    in_specs=[
        pl.BlockSpec((NCORE, BN, HH), lambda i: (0, i, 0)),
        pl.BlockSpec((BN, H), lambda i: (i, 0)),
        pl.BlockSpec((BN, 1), lambda i: (i, 0)),
        pl.BlockSpec((1, H), lambda i: (0, 0)),
        pl.BlockSpec((BN, 1), lambda i: (i, 0)),
    ],
    out_specs=pl.BlockSpec((G, H + 1), lambda i: (0, 0)),
    out_shape=jax.ShapeDtypeStruct((G, H + 1), jnp.float32),
)


def _head_body(p_ref, w_ref, b_ref, o_ref):
    P = p_ref[...]
    cnt = jnp.maximum(P[:, H:H + 1], 1.0)
    pooled = P[:, :H] / cnt
    logits = jnp.dot(pooled, w_ref[...],
                     preferred_element_type=jnp.float32) + b_ref[...]
    m = jnp.max(logits, axis=1, keepdims=True)
    e = jnp.exp(logits - m)
    lse = jnp.log(jnp.sum(e, axis=1, keepdims=True)) + m
    o_ref[...] = logits - lse


_head = pl.pallas_call(
    _head_body,
    out_shape=jax.ShapeDtypeStruct((G, C_OUT), jnp.float32),
)


def kernel(x, edge_index, batch, W1, b1, W2, b2, W3, b3, W4, b4, W5, b5,
           Wout, bout):
    src = edge_index[0]
    dst = edge_index[1]
    pad = E_PAD - E
    srcp = jnp.concatenate(
        [src, jnp.zeros((pad,), jnp.int32)]).reshape(EROWS, CHUNK)
    dstp = jnp.concatenate(
        [dst, N + (jnp.arange(pad, dtype=jnp.int32) % (N_ACC - N))]
    ).reshape(EROWS, CHUNK)

    degp = _deg_kernel(dstp)
    degsum = (1.0 + degp[0, :N] + degp[1, :N]).reshape(N, 1)
    batch2 = batch.reshape(N, 1)

    u1 = _pre(x, degsum)
    T = _scatter8_kernel(u1, srcp, dstp)
    g = _mmA(T, u1, degsum, b1.reshape(1, H), W1, W2)
    for (W, b) in ((W3, b2), (W4, b3), (W5, b4)):
        S = _scatter_kernel(g.reshape(2 * N, HH), srcp, dstp)
        g = _mml(S, g, degsum, b.reshape(1, H), W)
    S = _scatter_kernel(g.reshape(2 * N, HH), srcp, dstp)
    P = _pool(S, g, degsum, b5.reshape(1, H), batch2)
    return _head(P, Wout, bout.reshape(1, C_OUT))


# speedup vs baseline: 1.6523x; 1.6505x over previous
import functools

import jax
import jax.numpy as jnp
from jax import lax
from jax.experimental import pallas as pl
from jax.experimental.pallas import tpu as pltpu
from jax.experimental.pallas import tpu_sc as plsc

N = 50000
E = 800000
F_IN = 8
H = 64
HH = 32
C_OUT = 10
G = 128

NCORE = 2
NSUB = 16
CHUNK = 128
CPB = 8
NBLK = 50
NBLK1 = 25
F_PAD = 16
EPT = CHUNK * CPB * NBLK
E_PAD = EPT * NSUB
EROWS = E_PAD // CHUNK
RPT_E = CPB * NBLK

N_ACC = 50048
RPT = N_ACC // NSUB
ZROWS = 136
WAVE = 4

DEG_PAD = 51200
DPT = DEG_PAD // NSUB

BN = 5000

_MESH = plsc.VectorSubcoreMesh(
    core_axis_name="c", subcore_axis_name="s",
    num_cores=NCORE, num_subcores=NSUB,
)

def _z16():
    return jnp.zeros((16,), jnp.float32)


@functools.partial(
    pl.kernel,
    out_type=jax.ShapeDtypeStruct((NCORE, DEG_PAD), jnp.float32),
    mesh=_MESH,
    compiler_params=pltpu.CompilerParams(use_tc_tiling_on_sc=False),
    scratch_types=[
        pltpu.VMEM_SHARED((DEG_PAD,), jnp.float32),
        pltpu.VMEM((CPB, CHUNK), jnp.int32),
        pltpu.VMEM((CHUNK,), jnp.float32),
        pltpu.VMEM((DPT,), jnp.float32),
    ],
)
def _deg_kernel(dstp, out, acc, dbuf, ones, zbuf):
    c = lax.axis_index("c")
    s = lax.axis_index("s")

    def zfill(i, carry):
        zbuf[pl.ds(i * 16, 16)] = _z16()
        return carry
    lax.fori_loop(0, DPT // 16, zfill, 0)
    pltpu.sync_copy(zbuf, acc.at[pl.ds(s * DPT, DPT)])

    def ofill(i, carry):
        ones[pl.ds(i * 16, 16)] = _z16() + 1.0
        return carry
    lax.fori_loop(0, CHUNK // 16, ofill, 0)
    plsc.subcore_barrier()

    rb0 = (c * NSUB + s) * (NBLK1 * CPB)

    def block(b, carry):
        pltpu.sync_copy(dstp.at[pl.ds(rb0 + b * CPB, CPB)], dbuf)
        for j in range(CPB):
            pltpu.sync_copy(ones, acc.at[dbuf.at[j]], add=True)
        return carry

    lax.fori_loop(0, NBLK1, block, 0)
    plsc.subcore_barrier()
    pltpu.sync_copy(acc.at[pl.ds(s * DPT, DPT)], out.at[c, pl.ds(s * DPT, DPT)])


@functools.partial(
    pl.kernel,
    out_type=jax.ShapeDtypeStruct((NCORE, N_ACC, F_PAD), jnp.float32),
    mesh=_MESH,
    compiler_params=pltpu.CompilerParams(use_tc_tiling_on_sc=False),
    scratch_types=[
        pltpu.VMEM_SHARED((N_ACC, F_PAD), jnp.float32),
        pltpu.VMEM((CPB, CHUNK), jnp.int32),
        pltpu.VMEM((CPB, CHUNK), jnp.int32),
        pltpu.VMEM((CPB * CHUNK, F_PAD), jnp.float32),
        pltpu.VMEM((ZROWS, F_PAD), jnp.float32),
        pltpu.SemaphoreType.DMA,
    ],
)
def _scatter8_kernel(u, srcp, dstp, out, acc, sbuf, dbuf, rows, zbuf, sem):
    c = lax.axis_index("c")
    s = lax.axis_index("s")

    def zrow(r, carry):
        zbuf[r, pl.ds(0, 16)] = _z16()
        return carry
    lax.fori_loop(0, ZROWS, zrow, 0)
    row0 = s * RPT
    for jz in range(RPT // ZROWS):
        pltpu.sync_copy(zbuf, acc.at[pl.ds(row0 + jz * ZROWS, ZROWS)])
    plsc.subcore_barrier()

    rb0 = (c * NSUB + s) * (NBLK1 * CPB)

    def block(b, carry):
        rb = rb0 + b * CPB
        pltpu.sync_copy(srcp.at[pl.ds(rb, CPB)], sbuf)
        pltpu.sync_copy(dstp.at[pl.ds(rb, CPB)], dbuf)
        handles = []
        for j in range(CPB):
            handles.append(pltpu.async_copy(
                u.at[sbuf.at[j]], rows.at[pl.ds(j * CHUNK, CHUNK)], sem))
        for h in handles:
            h.wait()
        for j in range(CPB):
            pltpu.sync_copy(rows.at[pl.ds(j * CHUNK, CHUNK)],
                            acc.at[dbuf.at[j]], add=True)
        return carry

    lax.fori_loop(0, NBLK1, block, 0)
    plsc.subcore_barrier()
    pltpu.sync_copy(acc.at[pl.ds(row0, RPT)], out.at[c, pl.ds(row0, RPT)])


@functools.partial(
    pl.kernel,
    out_type=jax.ShapeDtypeStruct((NCORE, N_ACC, HH), jnp.float32),
    mesh=_MESH,
    compiler_params=pltpu.CompilerParams(use_tc_tiling_on_sc=False),
    scratch_types=[
        pltpu.VMEM_SHARED((N_ACC, HH), jnp.float32),
        pltpu.VMEM((CPB, CHUNK), jnp.int32),
        pltpu.VMEM((CPB, CHUNK), jnp.int32),
        pltpu.VMEM((CPB, CHUNK), jnp.int32),
        pltpu.VMEM((WAVE * CHUNK, HH), jnp.float32),
        pltpu.VMEM((ZROWS, HH), jnp.float32),
        pltpu.SemaphoreType.DMA,
    ],
)
def _scatter_kernel(g2, srcp, dstp, out, acc, sbuf, gbuf, dbuf, rows, zbuf, sem):
    c = lax.axis_index("c")
    s = lax.axis_index("s")

    def zrow(r, carry):
        zbuf[r, pl.ds(0, 16)] = _z16()
        zbuf[r, pl.ds(16, 16)] = _z16()
        return carry
    lax.fori_loop(0, ZROWS, zrow, 0)
    row0 = s * RPT
    for jz in range(RPT // ZROWS):
        pltpu.sync_copy(zbuf, acc.at[pl.ds(row0 + jz * ZROWS, ZROWS)])
    plsc.subcore_barrier()

    rb0 = s * RPT_E

    def block(b, carry):
        rb = rb0 + b * CPB
        pltpu.sync_copy(srcp.at[pl.ds(rb, CPB)], sbuf)
        pltpu.sync_copy(dstp.at[pl.ds(rb, CPB)], dbuf)

        def xf(i, carry2):
            r = i // 8
            k = (i % 8) * 16
            v = sbuf[r, pl.ds(k, 16)]
            gbuf[r, pl.ds(k, 16)] = v + v + c
            return carry2
        lax.fori_loop(0, CPB * 8, xf, 0)

        for w in range(CPB // WAVE):
            handles = []
            for j in range(WAVE):
                handles.append(pltpu.async_copy(
                    g2.at[gbuf.at[w * WAVE + j]],
                    rows.at[pl.ds(j * CHUNK, CHUNK)], sem))
            for h in handles:
                h.wait()
            for j in range(WAVE):
                pltpu.sync_copy(rows.at[pl.ds(j * CHUNK, CHUNK)],
                                acc.at[dbuf.at[w * WAVE + j]], add=True)
        return carry

    lax.fori_loop(0, NBLK, block, 0)
    plsc.subcore_barrier()
    pltpu.sync_copy(acc.at[pl.ds(row0, RPT)], out.at[c, pl.ds(row0, RPT)])


def _pre_body(x_ref, dg_ref, o_ref):
    d = lax.rsqrt(dg_ref[...])
    o_ref[...] = jnp.concatenate(
        [x_ref[...] * d, jnp.zeros((BN, F_PAD - F_IN), jnp.float32)], axis=1)


_pre = pl.pallas_call(
    _pre_body,
    grid=(N // BN,),
    in_specs=[
        pl.BlockSpec((BN, F_IN), lambda i: (i, 0)),
        pl.BlockSpec((BN, 1), lambda i: (i, 0)),
    ],
    out_specs=pl.BlockSpec((BN, F_PAD), lambda i: (i, 0)),
    out_shape=jax.ShapeDtypeStruct((N, F_PAD), jnp.float32),
)


def _mmA_body(t_ref, u_ref, dg_ref, b1_ref, w1_ref, w2_ref, o_ref):
    d = lax.rsqrt(dg_ref[...])
    t = t_ref[0] + t_ref[1] + u_ref[...]
    h = jnp.maximum(
        d * jnp.dot(t[:, :F_IN], w1_ref[...],
                    preferred_element_type=jnp.float32) + b1_ref[...], 0.0)
    o_ref[...] = jnp.dot(h * d, w2_ref[...], preferred_element_type=jnp.float32)


_mmA = pl.pallas_call(
    _mmA_body,
    grid=(N // BN,),
    in_specs=[
        pl.BlockSpec((NCORE, BN, F_PAD), lambda i: (0, i, 0)),
        pl.BlockSpec((BN, F_PAD), lambda i: (i, 0)),
        pl.BlockSpec((BN, 1), lambda i: (i, 0)),
        pl.BlockSpec((1, H), lambda i: (0, 0)),
        pl.BlockSpec((F_IN, H), lambda i: (0, 0)),
        pl.BlockSpec((H, H), lambda i: (0, 0)),
    ],
    out_specs=pl.BlockSpec((BN, H), lambda i: (i, 0)),
    out_shape=jax.ShapeDtypeStruct((N, H), jnp.float32),
)


def _mml_body(s_ref, g_ref, dg_ref, b_ref, w_ref, o_ref):
    d = lax.rsqrt(dg_ref[...])
    sb = jnp.concatenate([s_ref[0], s_ref[1]], axis=1)
    h = jnp.maximum(d * (sb + g_ref[...]) + b_ref[...], 0.0)
    o_ref[...] = jnp.dot(h * d, w_ref[...], preferred_element_type=jnp.float32)


_mml = pl.pallas_call(
    _mml_body,
    grid=(N // BN,),
    in_specs=[
        pl.BlockSpec((NCORE, BN, HH), lambda i: (0, i, 0)),
        pl.BlockSpec((BN, H), lambda i: (i, 0)),
        pl.BlockSpec((BN, 1), lambda i: (i, 0)),
        pl.BlockSpec((1, H), lambda i: (0, 0)),
        pl.BlockSpec((H, H), lambda i: (0, 0)),
    ],
    out_specs=pl.BlockSpec((BN, H), lambda i: (i, 0)),
    out_shape=jax.ShapeDtypeStruct((N, H), jnp.float32),
)


def _pool_body(s_ref, g_ref, dg_ref, b_ref, batch_ref, p_ref):
    i = pl.program_id(0)
    d = lax.rsqrt(dg_ref[...])
    sb = jnp.concatenate([s_ref[0], s_ref[1]], axis=1)
    h = jnp.maximum(d * (sb + g_ref[...]) + b_ref[...], 0.0)
    hh = jnp.concatenate([h, jnp.ones((BN, 1), jnp.float32)], axis=1)
    oh = (batch_ref[...] ==
          lax.broadcasted_iota(jnp.int32, (BN, G), 1)).astype(jnp.float32)
    contrib = lax.dot_general(oh, hh, (((0,), (0,)), ((), ())),
                              preferred_element_type=jnp.float32)

    @pl.when(i == 0)
    def _():
        p_ref[...] = contrib

    @pl.when(i != 0)
    def _():
        p_ref[...] = p_ref[...] + contrib


_pool = pl.pallas_call(
    _pool_body,
    grid=(N // BN,),
    in_specs=[
        pl.BlockSpec((NCORE, BN, HH), lambda i: (0, i, 0)),
        pl.BlockSpec((BN, H), lambda i: (i, 0)),
        pl.BlockSpec((BN, 1), lambda i: (i, 0)),
        pl.BlockSpec((1, H), lambda i: (0, 0)),
        pl.BlockSpec((BN, 1), lambda i: (i, 0)),
    ],
    out_specs=pl.BlockSpec((G, H + 1), lambda i: (0, 0)),
    out_shape=jax.ShapeDtypeStruct((G, H + 1), jnp.float32),
)


def _head_body(p_ref, w_ref, b_ref, o_ref):
    P = p_ref[...]
    cnt = jnp.maximum(P[:, H:H + 1], 1.0)
    pooled = P[:, :H] / cnt
    logits = jnp.dot(pooled, w_ref[...],
                     preferred_element_type=jnp.float32) + b_ref[...]
    m = jnp.max(logits, axis=1, keepdims=True)
    e = jnp.exp(logits - m)
    lse = jnp.log(jnp.sum(e, axis=1, keepdims=True)) + m
    o_ref[...] = logits - lse


_head = pl.pallas_call(
    _head_body,
    out_shape=jax.ShapeDtypeStruct((G, C_OUT), jnp.float32),
)


def kernel(x, edge_index, batch, W1, b1, W2, b2, W3, b3, W4, b4, W5, b5,
           Wout, bout):
    src = edge_index[0]
    dst = edge_index[1]
    pad = E_PAD - E
    srcp = jnp.concatenate(
        [src, jnp.arange(pad, dtype=jnp.int32) % N]).reshape(EROWS, CHUNK)
    dstp = jnp.concatenate(
        [dst, N + (jnp.arange(pad, dtype=jnp.int32) % (N_ACC - N))]
    ).reshape(EROWS, CHUNK)

    degp = _deg_kernel(dstp)
    degsum = (1.0 + degp[0, :N] + degp[1, :N]).reshape(N, 1)
    batch2 = batch.reshape(N, 1)

    u1 = _pre(x, degsum)
    T = _scatter8_kernel(u1, srcp, dstp)
    g = _mmA(T, u1, degsum, b1.reshape(1, H), W1, W2)
    for (W, b) in ((W3, b2), (W4, b3), (W5, b4)):
        S = _scatter_kernel(g.reshape(2 * N, HH), srcp, dstp)
        g = _mml(S, g, degsum, b.reshape(1, H), W)
    S = _scatter_kernel(g.reshape(2 * N, HH), srcp, dstp)
    P = _pool(S, g, degsum, b5.reshape(1, H), batch2)
    return _head(P, Wout, bout.reshape(1, C_OUT))


# speedup vs baseline: 1.9001x; 1.1500x over previous
import functools

import jax
import jax.numpy as jnp
from jax import lax
from jax.experimental import pallas as pl
from jax.experimental.pallas import tpu as pltpu
from jax.experimental.pallas import tpu_sc as plsc

N = 50000
E = 800000
F_IN = 8
H = 64
HH = 32
C_OUT = 10
G = 128

NCORE = 2
NSUB = 16
CHUNK = 128
CPB = 8
NBLK = 50
NBLK1 = 25
F_PAD = 16
EPT = CHUNK * CPB * NBLK
E_PAD = EPT * NSUB
EROWS = E_PAD // CHUNK
RPT_E = CPB * NBLK

N_ACC = 50048
RPT = N_ACC // NSUB
ZROWS = 136
WAVE = 4

DEG_PAD = 51200
DPT = DEG_PAD // NSUB

BN = 5000

_MESH = plsc.VectorSubcoreMesh(
    core_axis_name="c", subcore_axis_name="s",
    num_cores=NCORE, num_subcores=NSUB,
)

def _z16():
    return jnp.zeros((16,), jnp.float32)


@functools.partial(
    pl.kernel,
    out_type=jax.ShapeDtypeStruct((NCORE, DEG_PAD), jnp.float32),
    mesh=_MESH,
    compiler_params=pltpu.CompilerParams(use_tc_tiling_on_sc=False),
    scratch_types=[
        pltpu.VMEM_SHARED((DEG_PAD,), jnp.float32),
        pltpu.VMEM((CPB, CHUNK), jnp.int32),
        pltpu.VMEM((CHUNK,), jnp.float32),
        pltpu.VMEM((DPT,), jnp.float32),
    ],
)
def _deg_kernel(dstp, out, acc, dbuf, ones, zbuf):
    c = lax.axis_index("c")
    s = lax.axis_index("s")

    def zfill(i, carry):
        zbuf[pl.ds(i * 16, 16)] = _z16()
        return carry
    lax.fori_loop(0, DPT // 16, zfill, 0)
    pltpu.sync_copy(zbuf, acc.at[pl.ds(s * DPT, DPT)])

    def ofill(i, carry):
        ones[pl.ds(i * 16, 16)] = _z16() + 1.0
        return carry
    lax.fori_loop(0, CHUNK // 16, ofill, 0)
    plsc.subcore_barrier()

    rb0 = (c * NSUB + s) * (NBLK1 * CPB)

    def block(b, carry):
        pltpu.sync_copy(dstp.at[pl.ds(rb0 + b * CPB, CPB)], dbuf)
        for j in range(CPB):
            pltpu.sync_copy(ones, acc.at[dbuf.at[j]], add=True)
        return carry

    lax.fori_loop(0, NBLK1, block, 0)
    plsc.subcore_barrier()
    pltpu.sync_copy(acc.at[pl.ds(s * DPT, DPT)], out.at[c, pl.ds(s * DPT, DPT)])


@functools.partial(
    pl.kernel,
    out_type=jax.ShapeDtypeStruct((NCORE, N_ACC, F_PAD), jnp.float32),
    mesh=_MESH,
    compiler_params=pltpu.CompilerParams(use_tc_tiling_on_sc=False),
    scratch_types=[
        pltpu.VMEM_SHARED((N_ACC, F_PAD), jnp.float32),
        pltpu.VMEM((CPB, CHUNK), jnp.int32),
        pltpu.VMEM((CPB, CHUNK), jnp.int32),
        pltpu.VMEM((WAVE * CHUNK, F_PAD), jnp.float32),
        pltpu.VMEM((ZROWS, F_PAD), jnp.float32),
        pltpu.SemaphoreType.DMA,
        pltpu.SemaphoreType.DMA,
        pltpu.SemaphoreType.DMA,
        pltpu.SemaphoreType.DMA,
        pltpu.SemaphoreType.DMA,
        pltpu.SemaphoreType.DMA,
        pltpu.SemaphoreType.DMA,
        pltpu.SemaphoreType.DMA,
    ],
)
def _scatter8_kernel(u, srcp, dstp, out, acc, sbuf, dbuf, rows, zbuf,
                     g0, g1, g2s, g3, s0, s1, s2, s3):
    gsem = [g0, g1, g2s, g3]
    ssem = [s0, s1, s2, s3]
    c = lax.axis_index("c")
    s = lax.axis_index("s")

    def zrow(r, carry):
        zbuf[r, pl.ds(0, 16)] = _z16()
        return carry
    lax.fori_loop(0, ZROWS, zrow, 0)
    row0 = s * RPT
    for jz in range(RPT // ZROWS):
        pltpu.sync_copy(zbuf, acc.at[pl.ds(row0 + jz * ZROWS, ZROWS)])
    plsc.subcore_barrier()

    rb0 = (c * NSUB + s) * (NBLK1 * CPB)

    def block(b, carry):
        rb = rb0 + b * CPB
        pltpu.sync_copy(srcp.at[pl.ds(rb, CPB)], sbuf)
        pltpu.sync_copy(dstp.at[pl.ds(rb, CPB)], dbuf)
        gh = {}
        sh = {}
        for j in range(WAVE):
            gh[j] = pltpu.async_copy(
                u.at[sbuf.at[j]], rows.at[pl.ds(j * CHUNK, CHUNK)], gsem[j])
        for j in range(CPB):
            sl = j % WAVE
            gh[j].wait()
            sh[j] = pltpu.async_copy(
                rows.at[pl.ds(sl * CHUNK, CHUNK)], acc.at[dbuf.at[j]],
                ssem[sl], add=True)
            nj = j + WAVE
            if nj < CPB:
                sh[j].wait()
                gh[nj] = pltpu.async_copy(
                    u.at[sbuf.at[nj]], rows.at[pl.ds(sl * CHUNK, CHUNK)],
                    gsem[sl])
        for j in range(CPB - WAVE, CPB):
            sh[j].wait()
        return carry

    lax.fori_loop(0, NBLK1, block, 0)
    plsc.subcore_barrier()
    pltpu.sync_copy(acc.at[pl.ds(row0, RPT)], out.at[c, pl.ds(row0, RPT)])


@functools.partial(
    pl.kernel,
    out_type=jax.ShapeDtypeStruct((NCORE, N_ACC, HH), jnp.float32),
    mesh=_MESH,
    compiler_params=pltpu.CompilerParams(use_tc_tiling_on_sc=False),
    scratch_types=[
        pltpu.VMEM_SHARED((N_ACC, HH), jnp.float32),
        pltpu.VMEM((CPB, CHUNK), jnp.int32),
        pltpu.VMEM((CPB, CHUNK), jnp.int32),
        pltpu.VMEM((CPB, CHUNK), jnp.int32),
        pltpu.VMEM((WAVE * CHUNK, HH), jnp.float32),
        pltpu.VMEM((ZROWS, HH), jnp.float32),
        pltpu.SemaphoreType.DMA,
        pltpu.SemaphoreType.DMA,
        pltpu.SemaphoreType.DMA,
        pltpu.SemaphoreType.DMA,
        pltpu.SemaphoreType.DMA,
        pltpu.SemaphoreType.DMA,
        pltpu.SemaphoreType.DMA,
        pltpu.SemaphoreType.DMA,
    ],
)
def _scatter_kernel(g2, srcp, dstp, out, acc, sbuf, gbuf, dbuf, rows, zbuf,
                    g0, g1, g2s, g3, s0, s1, s2, s3):
    gsem = [g0, g1, g2s, g3]
    ssem = [s0, s1, s2, s3]
    c = lax.axis_index("c")
    s = lax.axis_index("s")

    def zrow(r, carry):
        zbuf[r, pl.ds(0, 16)] = _z16()
        zbuf[r, pl.ds(16, 16)] = _z16()
        return carry
    lax.fori_loop(0, ZROWS, zrow, 0)
    row0 = s * RPT
    for jz in range(RPT // ZROWS):
        pltpu.sync_copy(zbuf, acc.at[pl.ds(row0 + jz * ZROWS, ZROWS)])
    plsc.subcore_barrier()

    rb0 = s * RPT_E

    def block(b, carry):
        rb = rb0 + b * CPB
        pltpu.sync_copy(srcp.at[pl.ds(rb, CPB)], sbuf)
        pltpu.sync_copy(dstp.at[pl.ds(rb, CPB)], dbuf)

        def xf(i, carry2):
            r = i // 8
            k = (i % 8) * 16
            v = sbuf[r, pl.ds(k, 16)]
            gbuf[r, pl.ds(k, 16)] = v + v + c
            return carry2
        lax.fori_loop(0, CPB * 8, xf, 0)

        gh = {}
        sh = {}
        for j in range(WAVE):
            gh[j] = pltpu.async_copy(
                g2.at[gbuf.at[j]], rows.at[pl.ds(j * CHUNK, CHUNK)], gsem[j])
        for j in range(CPB):
            sl = j % WAVE
            gh[j].wait()
            sh[j] = pltpu.async_copy(
                rows.at[pl.ds(sl * CHUNK, CHUNK)], acc.at[dbuf.at[j]],
                ssem[sl], add=True)
            nj = j + WAVE
            if nj < CPB:
                sh[j].wait()
                gh[nj] = pltpu.async_copy(
                    g2.at[gbuf.at[nj]], rows.at[pl.ds(sl * CHUNK, CHUNK)],
                    gsem[sl])
        for j in range(CPB - WAVE, CPB):
            sh[j].wait()
        return carry

    lax.fori_loop(0, NBLK, block, 0)
    plsc.subcore_barrier()
    pltpu.sync_copy(acc.at[pl.ds(row0, RPT)], out.at[c, pl.ds(row0, RPT)])


def _pre_body(x_ref, dg_ref, o_ref):
    d = lax.rsqrt(dg_ref[...])
    o_ref[...] = jnp.concatenate(
        [x_ref[...] * d, jnp.zeros((BN, F_PAD - F_IN), jnp.float32)], axis=1)


_pre = pl.pallas_call(
    _pre_body,
    grid=(N // BN,),
    in_specs=[
        pl.BlockSpec((BN, F_IN), lambda i: (i, 0)),
        pl.BlockSpec((BN, 1), lambda i: (i, 0)),
    ],
    out_specs=pl.BlockSpec((BN, F_PAD), lambda i: (i, 0)),
    out_shape=jax.ShapeDtypeStruct((N, F_PAD), jnp.float32),
)


def _mmA_body(t_ref, u_ref, dg_ref, b1_ref, w1_ref, w2_ref, o_ref):
    d = lax.rsqrt(dg_ref[...])
    t = t_ref[0] + t_ref[1] + u_ref[...]
    h = jnp.maximum(
        d * jnp.dot(t[:, :F_IN], w1_ref[...],
                    preferred_element_type=jnp.float32) + b1_ref[...], 0.0)
    o_ref[...] = jnp.dot(h * d, w2_ref[...], preferred_element_type=jnp.float32)


_mmA = pl.pallas_call(
    _mmA_body,
    grid=(N // BN,),
    in_specs=[
        pl.BlockSpec((NCORE, BN, F_PAD), lambda i: (0, i, 0)),
        pl.BlockSpec((BN, F_PAD), lambda i: (i, 0)),
        pl.BlockSpec((BN, 1), lambda i: (i, 0)),
        pl.BlockSpec((1, H), lambda i: (0, 0)),
        pl.BlockSpec((F_IN, H), lambda i: (0, 0)),
        pl.BlockSpec((H, H), lambda i: (0, 0)),
    ],
    out_specs=pl.BlockSpec((BN, H), lambda i: (i, 0)),
    out_shape=jax.ShapeDtypeStruct((N, H), jnp.float32),
)


def _mml_body(s_ref, g_ref, dg_ref, b_ref, w_ref, o_ref):
    d = lax.rsqrt(dg_ref[...])
    sb = jnp.concatenate([s_ref[0], s_ref[1]], axis=1)
    h = jnp.maximum(d * (sb + g_ref[...]) + b_ref[...], 0.0)
    o_ref[...] = jnp.dot(h * d, w_ref[...], preferred_element_type=jnp.float32)


_mml = pl.pallas_call(
    _mml_body,
    grid=(N // BN,),
    in_specs=[
        pl.BlockSpec((NCORE, BN, HH), lambda i: (0, i, 0)),
        pl.BlockSpec((BN, H), lambda i: (i, 0)),
        pl.BlockSpec((BN, 1), lambda i: (i, 0)),
        pl.BlockSpec((1, H), lambda i: (0, 0)),
        pl.BlockSpec((H, H), lambda i: (0, 0)),
    ],
    out_specs=pl.BlockSpec((BN, H), lambda i: (i, 0)),
    out_shape=jax.ShapeDtypeStruct((N, H), jnp.float32),
)


def _pool_body(s_ref, g_ref, dg_ref, b_ref, batch_ref, p_ref):
    i = pl.program_id(0)
    d = lax.rsqrt(dg_ref[...])
    sb = jnp.concatenate([s_ref[0], s_ref[1]], axis=1)
    h = jnp.maximum(d * (sb + g_ref[...]) + b_ref[...], 0.0)
    hh = jnp.concatenate([h, jnp.ones((BN, 1), jnp.float32)], axis=1)
    oh = (batch_ref[...] ==
          lax.broadcasted_iota(jnp.int32, (BN, G), 1)).astype(jnp.float32)
    contrib = lax.dot_general(oh, hh, (((0,), (0,)), ((), ())),
                              preferred_element_type=jnp.float32)

    @pl.when(i == 0)
    def _():
        p_ref[...] = contrib

    @pl.when(i != 0)
    def _():
        p_ref[...] = p_ref[...] + contrib


_pool = pl.pallas_call(
    _pool_body,
    grid=(N // BN,),
    in_specs=[
        pl.BlockSpec((NCORE, BN, HH), lambda i: (0, i, 0)),
        pl.BlockSpec((BN, H), lambda i: (i, 0)),
        pl.BlockSpec((BN, 1), lambda i: (i, 0)),
        pl.BlockSpec((1, H), lambda i: (0, 0)),
        pl.BlockSpec((BN, 1), lambda i: (i, 0)),
    ],
    out_specs=pl.BlockSpec((G, H + 1), lambda i: (0, 0)),
    out_shape=jax.ShapeDtypeStruct((G, H + 1), jnp.float32),
)


def _head_body(p_ref, w_ref, b_ref, o_ref):
    P = p_ref[...]
    cnt = jnp.maximum(P[:, H:H + 1], 1.0)
    pooled = P[:, :H] / cnt
    logits = jnp.dot(pooled, w_ref[...],
                     preferred_element_type=jnp.float32) + b_ref[...]
    m = jnp.max(logits, axis=1, keepdims=True)
    e = jnp.exp(logits - m)
    lse = jnp.log(jnp.sum(e, axis=1, keepdims=True)) + m
    o_ref[...] = logits - lse


_head = pl.pallas_call(
    _head_body,
    out_shape=jax.ShapeDtypeStruct((G, C_OUT), jnp.float32),
)


def kernel(x, edge_index, batch, W1, b1, W2, b2, W3, b3, W4, b4, W5, b5,
           Wout, bout):
    src = edge_index[0]
    dst = edge_index[1]
    pad = E_PAD - E
    srcp = jnp.concatenate(
        [src, jnp.arange(pad, dtype=jnp.int32) % N]).reshape(EROWS, CHUNK)
    dstp = jnp.concatenate(
        [dst, N + (jnp.arange(pad, dtype=jnp.int32) % (N_ACC - N))]
    ).reshape(EROWS, CHUNK)

    degp = _deg_kernel(dstp)
    degsum = (1.0 + degp[0, :N] + degp[1, :N]).reshape(N, 1)
    batch2 = batch.reshape(N, 1)

    u1 = _pre(x, degsum)
    T = _scatter8_kernel(u1, srcp, dstp)
    g = _mmA(T, u1, degsum, b1.reshape(1, H), W1, W2)
    for (W, b) in ((W3, b2), (W4, b3), (W5, b4)):
        S = _scatter_kernel(g.reshape(2 * N, HH), srcp, dstp)
        g = _mml(S, g, degsum, b.reshape(1, H), W)
    S = _scatter_kernel(g.reshape(2 * N, HH), srcp, dstp)
    P = _pool(S, g, degsum, b5.reshape(1, H), batch2)
    return _head(P, Wout, bout.reshape(1, C_OUT))


# speedup vs baseline: 2.1865x; 1.1507x over previous
import functools

import jax
import jax.numpy as jnp
from jax import lax
from jax.experimental import pallas as pl
from jax.experimental.pallas import tpu as pltpu
from jax.experimental.pallas import tpu_sc as plsc

N = 50000
E = 800000
F_IN = 8
H = 64
HH = 32
C_OUT = 10
G = 128

NCORE = 2
NSUB = 16
CHUNK = 128
CPB = 8
NBLK = 50
NBLK1 = 25
F_PAD = 16
EPT = CHUNK * CPB * NBLK
E_PAD = EPT * NSUB
EROWS = E_PAD // CHUNK
RPT_E = CPB * NBLK

N_ACC = 50048
RPT = N_ACC // NSUB
ZROWS = 136
WAVE = 4

DEG_PAD = 51200
DPT = DEG_PAD // NSUB

BN = 5000

_MESH = plsc.VectorSubcoreMesh(
    core_axis_name="c", subcore_axis_name="s",
    num_cores=NCORE, num_subcores=NSUB,
)

def _z16():
    return jnp.zeros((16,), jnp.float32)


@functools.partial(
    pl.kernel,
    out_type=jax.ShapeDtypeStruct((NCORE, DEG_PAD), jnp.float32),
    mesh=_MESH,
    compiler_params=pltpu.CompilerParams(use_tc_tiling_on_sc=False),
    scratch_types=[
        pltpu.VMEM_SHARED((DEG_PAD,), jnp.float32),
        pltpu.VMEM((CPB, CHUNK), jnp.int32),
        pltpu.VMEM((CHUNK,), jnp.float32),
        pltpu.VMEM((DPT,), jnp.float32),
    ],
)
def _deg_kernel(dstp, out, acc, dbuf, ones, zbuf):
    c = lax.axis_index("c")
    s = lax.axis_index("s")

    def zfill(i, carry):
        zbuf[pl.ds(i * 16, 16)] = _z16()
        return carry
    lax.fori_loop(0, DPT // 16, zfill, 0)
    pltpu.sync_copy(zbuf, acc.at[pl.ds(s * DPT, DPT)])

    def ofill(i, carry):
        ones[pl.ds(i * 16, 16)] = _z16() + 1.0
        return carry
    lax.fori_loop(0, CHUNK // 16, ofill, 0)
    plsc.subcore_barrier()

    rb0 = (c * NSUB + s) * (NBLK1 * CPB)

    def block(b, carry):
        pltpu.sync_copy(dstp.at[pl.ds(rb0 + b * CPB, CPB)], dbuf)
        for j in range(CPB):
            pltpu.sync_copy(ones, acc.at[dbuf.at[j]], add=True)
        return carry

    lax.fori_loop(0, NBLK1, block, 0)
    plsc.subcore_barrier()
    pltpu.sync_copy(acc.at[pl.ds(s * DPT, DPT)], out.at[c, pl.ds(s * DPT, DPT)])


@functools.partial(
    pl.kernel,
    out_type=jax.ShapeDtypeStruct((NCORE, N_ACC, F_PAD), jnp.float32),
    mesh=_MESH,
    compiler_params=pltpu.CompilerParams(use_tc_tiling_on_sc=False),
    scratch_types=[
        pltpu.VMEM_SHARED((N_ACC, F_PAD), jnp.float32),
        pltpu.VMEM((CPB, CHUNK), jnp.int32),
        pltpu.VMEM((CPB, CHUNK), jnp.int32),
        pltpu.VMEM((WAVE * CHUNK, F_PAD), jnp.float32),
        pltpu.VMEM((ZROWS, F_PAD), jnp.float32),
        pltpu.SemaphoreType.DMA,
        pltpu.SemaphoreType.DMA,
        pltpu.SemaphoreType.DMA,
        pltpu.SemaphoreType.DMA,
        pltpu.SemaphoreType.DMA,
        pltpu.SemaphoreType.DMA,
        pltpu.SemaphoreType.DMA,
        pltpu.SemaphoreType.DMA,
    ],
)
def _scatter8_kernel(u, srcp, dstp, out, acc, sbuf, dbuf, rows, zbuf,
                     g0, g1, g2s, g3, s0, s1, s2, s3):
    gsem = [g0, g1, g2s, g3]
    ssem = [s0, s1, s2, s3]
    c = lax.axis_index("c")
    s = lax.axis_index("s")

    def zrow(r, carry):
        zbuf[r, pl.ds(0, 16)] = _z16()
        return carry
    lax.fori_loop(0, ZROWS, zrow, 0)
    row0 = s * RPT
    for jz in range(RPT // ZROWS):
        pltpu.sync_copy(zbuf, acc.at[pl.ds(row0 + jz * ZROWS, ZROWS)])
    plsc.subcore_barrier()

    rb0 = (c * NSUB + s) * (NBLK1 * CPB)

    def block(b, carry):
        rb = rb0 + b * CPB
        pltpu.sync_copy(srcp.at[pl.ds(rb, CPB)], sbuf)
        pltpu.sync_copy(dstp.at[pl.ds(rb, CPB)], dbuf)
        gh = {}
        sh = {}
        for j in range(WAVE):
            gh[j] = pltpu.async_copy(
                u.at[sbuf.at[j]], rows.at[pl.ds(j * CHUNK, CHUNK)], gsem[j])
        for j in range(CPB):
            sl = j % WAVE
            gh[j].wait()
            sh[j] = pltpu.async_copy(
                rows.at[pl.ds(sl * CHUNK, CHUNK)], acc.at[dbuf.at[j]],
                ssem[sl], add=True)
            nj = j + WAVE
            if nj < CPB:
                sh[j].wait()
                gh[nj] = pltpu.async_copy(
                    u.at[sbuf.at[nj]], rows.at[pl.ds(sl * CHUNK, CHUNK)],
                    gsem[sl])
        for j in range(CPB - WAVE, CPB):
            sh[j].wait()
        return carry

    lax.fori_loop(0, NBLK1, block, 0)
    plsc.subcore_barrier()
    pltpu.sync_copy(acc.at[pl.ds(row0, RPT)], out.at[c, pl.ds(row0, RPT)])


@functools.partial(
    pl.kernel,
    out_type=jax.ShapeDtypeStruct((NCORE, N_ACC, HH), jnp.float32),
    mesh=_MESH,
    compiler_params=pltpu.CompilerParams(use_tc_tiling_on_sc=False),
    scratch_types=[
        pltpu.VMEM_SHARED((N_ACC, HH), jnp.float32),
        pltpu.VMEM((CPB, CHUNK), jnp.int32),
        pltpu.VMEM((CPB, CHUNK), jnp.int32),
        pltpu.VMEM((CPB, CHUNK), jnp.int32),
        pltpu.VMEM((CPB, CHUNK), jnp.int32),
        pltpu.VMEM((CPB, CHUNK), jnp.int32),
        pltpu.VMEM((CPB, CHUNK), jnp.int32),
        pltpu.VMEM((WAVE * CHUNK, HH), jnp.float32),
        pltpu.VMEM((ZROWS, HH), jnp.float32),
        pltpu.SemaphoreType.DMA,
        pltpu.SemaphoreType.DMA,
        pltpu.SemaphoreType.DMA,
        pltpu.SemaphoreType.DMA,
        pltpu.SemaphoreType.DMA,
        pltpu.SemaphoreType.DMA,
        pltpu.SemaphoreType.DMA,
        pltpu.SemaphoreType.DMA,
        pltpu.SemaphoreType.DMA,
        pltpu.SemaphoreType.DMA,
    ],
)
def _scatter_kernel(g2, srcp, dstp, out, acc, sbufA, gbufA, dbufA,
                    sbufB, gbufB, dbufB, rows, zbuf,
                    g0, g1, g2s, g3, s0, s1, s2, s3, isemA, isemB):
    gsem = [g0, g1, g2s, g3]
    ssem = [s0, s1, s2, s3]
    c = lax.axis_index("c")
    s = lax.axis_index("s")

    def zrow(r, carry):
        zbuf[r, pl.ds(0, 16)] = _z16()
        zbuf[r, pl.ds(16, 16)] = _z16()
        return carry
    lax.fori_loop(0, ZROWS, zrow, 0)
    row0 = s * RPT
    for jz in range(RPT // ZROWS):
        pltpu.sync_copy(zbuf, acc.at[pl.ds(row0 + jz * ZROWS, ZROWS)])
    plsc.subcore_barrier()

    rb0 = s * RPT_E

    def fire_idx(b, sb, db, isem):
        rb = rb0 + b * CPB
        pltpu.async_copy(srcp.at[pl.ds(rb, CPB)], sb, isem)
        pltpu.async_copy(dstp.at[pl.ds(rb, CPB)], db, isem)

    def wait_idx(sb, db, isem):
        pltpu.make_async_copy(srcp.at[pl.ds(0, CPB)], sb, isem).wait()
        pltpu.make_async_copy(dstp.at[pl.ds(0, CPB)], db, isem).wait()

    def run_block(sbuf, gbuf, dbuf):
        def xf(i, carry2):
            r = i // 8
            k = (i % 8) * 16
            v = sbuf[r, pl.ds(k, 16)]
            gbuf[r, pl.ds(k, 16)] = v + v + c
            return carry2
        lax.fori_loop(0, CPB * 8, xf, 0)

        gh = {}
        sh = {}
        for j in range(WAVE):
            gh[j] = pltpu.async_copy(
                g2.at[gbuf.at[j]], rows.at[pl.ds(j * CHUNK, CHUNK)], gsem[j])
        for j in range(CPB):
            sl = j % WAVE
            gh[j].wait()
            sh[j] = pltpu.async_copy(
                rows.at[pl.ds(sl * CHUNK, CHUNK)], acc.at[dbuf.at[j]],
                ssem[sl], add=True)
            nj = j + WAVE
            if nj < CPB:
                sh[j].wait()
                gh[nj] = pltpu.async_copy(
                    g2.at[gbuf.at[nj]], rows.at[pl.ds(sl * CHUNK, CHUNK)],
                    gsem[sl])
        for j in range(CPB - WAVE, CPB):
            sh[j].wait()

    fire_idx(0, sbufA, dbufA, isemA)

    def pair(t, carry):
        b0 = 2 * t
        fire_idx(b0 + 1, sbufB, dbufB, isemB)
        wait_idx(sbufA, dbufA, isemA)
        run_block(sbufA, gbufA, dbufA)

        @pl.when(t < NBLK // 2 - 1)
        def _():
            fire_idx(b0 + 2, sbufA, dbufA, isemA)
        wait_idx(sbufB, dbufB, isemB)
        run_block(sbufB, gbufB, dbufB)
        return carry

    lax.fori_loop(0, NBLK // 2, pair, 0)
    plsc.subcore_barrier()
    pltpu.sync_copy(acc.at[pl.ds(row0, RPT)], out.at[c, pl.ds(row0, RPT)])


def _pre_body(x_ref, dg_ref, o_ref):
    d = lax.rsqrt(dg_ref[...])
    o_ref[...] = jnp.concatenate(
        [x_ref[...] * d, jnp.zeros((BN, F_PAD - F_IN), jnp.float32)], axis=1)


_pre = pl.pallas_call(
    _pre_body,
    grid=(N // BN,),
    in_specs=[
        pl.BlockSpec((BN, F_IN), lambda i: (i, 0)),
        pl.BlockSpec((BN, 1), lambda i: (i, 0)),
    ],
    out_specs=pl.BlockSpec((BN, F_PAD), lambda i: (i, 0)),
    out_shape=jax.ShapeDtypeStruct((N, F_PAD), jnp.float32),
)


def _mmA_body(t_ref, u_ref, dg_ref, b1_ref, w1_ref, w2_ref, o_ref):
    d = lax.rsqrt(dg_ref[...])
    t = t_ref[0] + t_ref[1] + u_ref[...]
    h = jnp.maximum(
        d * jnp.dot(t[:, :F_IN], w1_ref[...],
                    preferred_element_type=jnp.float32) + b1_ref[...], 0.0)
    o_ref[...] = jnp.dot(h * d, w2_ref[...], preferred_element_type=jnp.float32)


_mmA = pl.pallas_call(
    _mmA_body,
    grid=(N // BN,),
    in_specs=[
        pl.BlockSpec((NCORE, BN, F_PAD), lambda i: (0, i, 0)),
        pl.BlockSpec((BN, F_PAD), lambda i: (i, 0)),
        pl.BlockSpec((BN, 1), lambda i: (i, 0)),
        pl.BlockSpec((1, H), lambda i: (0, 0)),
        pl.BlockSpec((F_IN, H), lambda i: (0, 0)),
        pl.BlockSpec((H, H), lambda i: (0, 0)),
    ],
    out_specs=pl.BlockSpec((BN, H), lambda i: (i, 0)),
    out_shape=jax.ShapeDtypeStruct((N, H), jnp.float32),
)


def _mml_body(s_ref, g_ref, dg_ref, b_ref, w_ref, o_ref):
    d = lax.rsqrt(dg_ref[...])
    sb = jnp.concatenate([s_ref[0], s_ref[1]], axis=1)
    h = jnp.maximum(d * (sb + g_ref[...]) + b_ref[...], 0.0)
    o_ref[...] = jnp.dot(h * d, w_ref[...], preferred_element_type=jnp.float32)


_mml = pl.pallas_call(
    _mml_body,
    grid=(N // BN,),
    in_specs=[
        pl.BlockSpec((NCORE, BN, HH), lambda i: (0, i, 0)),
        pl.BlockSpec((BN, H), lambda i: (i, 0)),
        pl.BlockSpec((BN, 1), lambda i: (i, 0)),
        pl.BlockSpec((1, H), lambda i: (0, 0)),
        pl.BlockSpec((H, H), lambda i: (0, 0)),
    ],
    out_specs=pl.BlockSpec((BN, H), lambda i: (i, 0)),
    out_shape=jax.ShapeDtypeStruct((N, H), jnp.float32),
)


def _pool_body(s_ref, g_ref, dg_ref, b_ref, batch_ref, p_ref):
    i = pl.program_id(0)
    d = lax.rsqrt(dg_ref[...])
    sb = jnp.concatenate([s_ref[0], s_ref[1]], axis=1)
    h = jnp.maximum(d * (sb + g_ref[...]) + b_ref[...], 0.0)
    hh = jnp.concatenate([h, jnp.ones((BN, 1), jnp.float32)], axis=1)
    oh = (batch_ref[...] ==
          lax.broadcasted_iota(jnp.int32, (BN, G), 1)).astype(jnp.float32)
    contrib = lax.dot_general(oh, hh, (((0,), (0,)), ((), ())),
                              preferred_element_type=jnp.float32)

    @pl.when(i == 0)
    def _():
        p_ref[...] = contrib

    @pl.when(i != 0)
    def _():
        p_ref[...] = p_ref[...] + contrib


_pool = pl.pallas_call(
    _pool_body,
    grid=(N // BN,),
    in_specs=[
        pl.BlockSpec((NCORE, BN, HH), lambda i: (0, i, 0)),
        pl.BlockSpec((BN, H), lambda i: (i, 0)),
        pl.BlockSpec((BN, 1), lambda i: (i, 0)),
        pl.BlockSpec((1, H), lambda i: (0, 0)),
        pl.BlockSpec((BN, 1), lambda i: (i, 0)),
    ],
    out_specs=pl.BlockSpec((G, H + 1), lambda i: (0, 0)),
    out_shape=jax.ShapeDtypeStruct((G, H + 1), jnp.float32),
)


def _head_body(p_ref, w_ref, b_ref, o_ref):
    P = p_ref[...]
    cnt = jnp.maximum(P[:, H:H + 1], 1.0)
    pooled = P[:, :H] / cnt
    logits = jnp.dot(pooled, w_ref[...],
                     preferred_element_type=jnp.float32) + b_ref[...]
    m = jnp.max(logits, axis=1, keepdims=True)
    e = jnp.exp(logits - m)
    lse = jnp.log(jnp.sum(e, axis=1, keepdims=True)) + m
    o_ref[...] = logits - lse


_head = pl.pallas_call(
    _head_body,
    out_shape=jax.ShapeDtypeStruct((G, C_OUT), jnp.float32),
)


def kernel(x, edge_index, batch, W1, b1, W2, b2, W3, b3, W4, b4, W5, b5,
           Wout, bout):
    src = edge_index[0]
    dst = edge_index[1]
    pad = E_PAD - E
    srcp = jnp.concatenate(
        [src, jnp.arange(pad, dtype=jnp.int32) % N]).reshape(EROWS, CHUNK)
    dstp = jnp.concatenate(
        [dst, N + (jnp.arange(pad, dtype=jnp.int32) % (N_ACC - N))]
    ).reshape(EROWS, CHUNK)

    degp = _deg_kernel(dstp)
    degsum = (1.0 + degp[0, :N] + degp[1, :N]).reshape(N, 1)
    batch2 = batch.reshape(N, 1)

    u1 = _pre(x, degsum)
    T = _scatter8_kernel(u1, srcp, dstp)
    g = _mmA(T, u1, degsum, b1.reshape(1, H), W1, W2)
    for (W, b) in ((W3, b2), (W4, b3), (W5, b4)):
        S = _scatter_kernel(g.reshape(2 * N, HH), srcp, dstp)
        g = _mml(S, g, degsum, b.reshape(1, H), W)
    S = _scatter_kernel(g.reshape(2 * N, HH), srcp, dstp)
    P = _pool(S, g, degsum, b5.reshape(1, H), batch2)
    return _head(P, Wout, bout.reshape(1, C_OUT))


# speedup vs baseline: 2.2559x; 1.0317x over previous
import functools

import jax
import jax.numpy as jnp
from jax import lax
from jax.experimental import pallas as pl
from jax.experimental.pallas import tpu as pltpu
from jax.experimental.pallas import tpu_sc as plsc

N = 50000
E = 800000
F_IN = 8
H = 64
HH = 32
C_OUT = 10
G = 128

NCORE = 2
NSUB = 16
CHUNK = 128
CPB = 8
NBLK = 50
NBLK1 = 25
F_PAD = 16
EPT = CHUNK * CPB * NBLK
E_PAD = EPT * NSUB
EROWS = E_PAD // CHUNK
RPT_E = CPB * NBLK

N_ACC = 50048
RPT = N_ACC // NSUB
ZROWS = 136
WAVE = 4

DEG_PAD = 51200
DPT = DEG_PAD // NSUB

BN = 5000

_MESH = plsc.VectorSubcoreMesh(
    core_axis_name="c", subcore_axis_name="s",
    num_cores=NCORE, num_subcores=NSUB,
)

def _z16():
    return jnp.zeros((16,), jnp.float32)


@functools.partial(
    pl.kernel,
    out_type=jax.ShapeDtypeStruct((NCORE, DEG_PAD), jnp.float32),
    mesh=_MESH,
    compiler_params=pltpu.CompilerParams(use_tc_tiling_on_sc=False),
    scratch_types=[
        pltpu.VMEM_SHARED((DEG_PAD,), jnp.float32),
        pltpu.VMEM((CPB, CHUNK), jnp.int32),
        pltpu.VMEM((CHUNK,), jnp.float32),
        pltpu.VMEM((DPT,), jnp.float32),
    ],
)
def _deg_kernel(dstp, out, acc, dbuf, ones, zbuf):
    c = lax.axis_index("c")
    s = lax.axis_index("s")

    def zfill(i, carry):
        zbuf[pl.ds(i * 16, 16)] = _z16()
        return carry
    lax.fori_loop(0, DPT // 16, zfill, 0)
    pltpu.sync_copy(zbuf, acc.at[pl.ds(s * DPT, DPT)])

    def ofill(i, carry):
        ones[pl.ds(i * 16, 16)] = _z16() + 1.0
        return carry
    lax.fori_loop(0, CHUNK // 16, ofill, 0)
    plsc.subcore_barrier()

    rb0 = (c * NSUB + s) * (NBLK1 * CPB)

    def block(b, carry):
        pltpu.sync_copy(dstp.at[pl.ds(rb0 + b * CPB, CPB)], dbuf)
        for j in range(CPB):
            pltpu.sync_copy(ones, acc.at[dbuf.at[j]], add=True)
        return carry

    lax.fori_loop(0, NBLK1, block, 0)
    plsc.subcore_barrier()
    pltpu.sync_copy(acc.at[pl.ds(s * DPT, DPT)], out.at[c, pl.ds(s * DPT, DPT)])


@functools.partial(
    pl.kernel,
    out_type=jax.ShapeDtypeStruct((NCORE, N_ACC, F_PAD), jnp.float32),
    mesh=_MESH,
    compiler_params=pltpu.CompilerParams(use_tc_tiling_on_sc=False),
    scratch_types=[
        pltpu.VMEM_SHARED((N_ACC, F_PAD), jnp.float32),
        pltpu.VMEM((CPB, CHUNK), jnp.int32),
        pltpu.VMEM((CPB, CHUNK), jnp.int32),
        pltpu.VMEM((WAVE * CHUNK, F_PAD), jnp.float32),
        pltpu.VMEM((ZROWS, F_PAD), jnp.float32),
        pltpu.SemaphoreType.DMA,
        pltpu.SemaphoreType.DMA,
        pltpu.SemaphoreType.DMA,
        pltpu.SemaphoreType.DMA,
        pltpu.SemaphoreType.DMA,
        pltpu.SemaphoreType.DMA,
        pltpu.SemaphoreType.DMA,
        pltpu.SemaphoreType.DMA,
    ],
)
def _scatter8_kernel(u, srcp, dstp, out, acc, sbuf, dbuf, rows, zbuf,
                     g0, g1, g2s, g3, s0, s1, s2, s3):
    gsem = [g0, g1, g2s, g3]
    ssem = [s0, s1, s2, s3]
    c = lax.axis_index("c")
    s = lax.axis_index("s")

    def zrow(r, carry):
        zbuf[r, pl.ds(0, 16)] = _z16()
        return carry
    lax.fori_loop(0, ZROWS, zrow, 0)
    row0 = s * RPT
    for jz in range(RPT // ZROWS):
        pltpu.sync_copy(zbuf, acc.at[pl.ds(row0 + jz * ZROWS, ZROWS)])
    plsc.subcore_barrier()

    rb0 = (c * NSUB + s) * (NBLK1 * CPB)

    def block(b, carry):
        rb = rb0 + b * CPB
        pltpu.sync_copy(srcp.at[pl.ds(rb, CPB)], sbuf)
        pltpu.sync_copy(dstp.at[pl.ds(rb, CPB)], dbuf)
        gh = {}
        sh = {}
        for j in range(WAVE):
            gh[j] = pltpu.async_copy(
                u.at[sbuf.at[j]], rows.at[pl.ds(j * CHUNK, CHUNK)], gsem[j])
        for j in range(CPB):
            sl = j % WAVE
            gh[j].wait()
            sh[j] = pltpu.async_copy(
                rows.at[pl.ds(sl * CHUNK, CHUNK)], acc.at[dbuf.at[j]],
                ssem[sl], add=True)
            nj = j + WAVE
            if nj < CPB:
                sh[j].wait()
                gh[nj] = pltpu.async_copy(
                    u.at[sbuf.at[nj]], rows.at[pl.ds(sl * CHUNK, CHUNK)],
                    gsem[sl])
        for j in range(CPB - WAVE, CPB):
            sh[j].wait()
        return carry

    lax.fori_loop(0, NBLK1, block, 0)
    plsc.subcore_barrier()
    pltpu.sync_copy(acc.at[pl.ds(row0, RPT)], out.at[c, pl.ds(row0, RPT)])


@functools.partial(
    pl.kernel,
    out_type=jax.ShapeDtypeStruct((N_ACC, H), jnp.float32),
    mesh=_MESH,
    compiler_params=pltpu.CompilerParams(use_tc_tiling_on_sc=False),
    scratch_types=[
        pltpu.VMEM_SHARED((N_ACC, HH), jnp.float32),
        pltpu.VMEM((CPB, CHUNK), jnp.int32),
        pltpu.VMEM((CPB, CHUNK), jnp.int32),
        pltpu.VMEM((CPB, CHUNK), jnp.int32),
        pltpu.VMEM((CPB, CHUNK), jnp.int32),
        pltpu.VMEM((CPB, CHUNK), jnp.int32),
        pltpu.VMEM((CPB, CHUNK), jnp.int32),
        pltpu.VMEM((WAVE * CHUNK, HH), jnp.float32),
        pltpu.VMEM((ZROWS, HH), jnp.float32),
        pltpu.SemaphoreType.DMA,
        pltpu.SemaphoreType.DMA,
        pltpu.SemaphoreType.DMA,
        pltpu.SemaphoreType.DMA,
        pltpu.SemaphoreType.DMA,
        pltpu.SemaphoreType.DMA,
        pltpu.SemaphoreType.DMA,
        pltpu.SemaphoreType.DMA,
        pltpu.SemaphoreType.DMA,
        pltpu.SemaphoreType.DMA,
    ],
)
def _scatter_kernel(g2, srcp, dstp, out, acc, sbufA, gbufA, dbufA,
                    sbufB, gbufB, dbufB, rows, zbuf,
                    g0, g1, g2s, g3, s0, s1, s2, s3, isemA, isemB):
    gsem = [g0, g1, g2s, g3]
    ssem = [s0, s1, s2, s3]
    c = lax.axis_index("c")
    s = lax.axis_index("s")

    def zrow(r, carry):
        zbuf[r, pl.ds(0, 16)] = _z16()
        zbuf[r, pl.ds(16, 16)] = _z16()
        return carry
    lax.fori_loop(0, ZROWS, zrow, 0)
    row0 = s * RPT
    for jz in range(RPT // ZROWS):
        pltpu.sync_copy(zbuf, acc.at[pl.ds(row0 + jz * ZROWS, ZROWS)])
    plsc.subcore_barrier()

    rb0 = s * RPT_E

    def fire_idx(b, sb, db, isem):
        rb = rb0 + b * CPB
        pltpu.async_copy(srcp.at[pl.ds(rb, CPB)], sb, isem)
        pltpu.async_copy(dstp.at[pl.ds(rb, CPB)], db, isem)

    def wait_idx(sb, db, isem):
        pltpu.make_async_copy(srcp.at[pl.ds(0, CPB)], sb, isem).wait()
        pltpu.make_async_copy(dstp.at[pl.ds(0, CPB)], db, isem).wait()

    def run_block(sbuf, gbuf, dbuf):
        def xf(i, carry2):
            r = i // 8
            k = (i % 8) * 16
            v = sbuf[r, pl.ds(k, 16)]
            gbuf[r, pl.ds(k, 16)] = v + v + c
            return carry2
        lax.fori_loop(0, CPB * 8, xf, 0)

        gh = {}
        sh = {}
        for j in range(WAVE):
            gh[j] = pltpu.async_copy(
                g2.at[gbuf.at[j]], rows.at[pl.ds(j * CHUNK, CHUNK)], gsem[j])
        for j in range(CPB):
            sl = j % WAVE
            gh[j].wait()
            sh[j] = pltpu.async_copy(
                rows.at[pl.ds(sl * CHUNK, CHUNK)], acc.at[dbuf.at[j]],
                ssem[sl], add=True)
            nj = j + WAVE
            if nj < CPB:
                sh[j].wait()
                gh[nj] = pltpu.async_copy(
                    g2.at[gbuf.at[nj]], rows.at[pl.ds(sl * CHUNK, CHUNK)],
                    gsem[sl])
        for j in range(CPB - WAVE, CPB):
            sh[j].wait()

    fire_idx(0, sbufA, dbufA, isemA)

    def pair(t, carry):
        b0 = 2 * t
        fire_idx(b0 + 1, sbufB, dbufB, isemB)
        wait_idx(sbufA, dbufA, isemA)
        run_block(sbufA, gbufA, dbufA)

        @pl.when(t < NBLK // 2 - 1)
        def _():
            fire_idx(b0 + 2, sbufA, dbufA, isemA)
        wait_idx(sbufB, dbufB, isemB)
        run_block(sbufB, gbufB, dbufB)
        return carry

    lax.fori_loop(0, NBLK // 2, pair, 0)
    plsc.subcore_barrier()
    pltpu.sync_copy(acc.at[pl.ds(row0, RPT)],
                    out.at[pl.ds(row0, RPT), pl.ds(c * HH, HH)])


def _pre_body(x_ref, dg_ref, o_ref):
    d = lax.rsqrt(dg_ref[...])
    o_ref[...] = jnp.concatenate(
        [x_ref[...] * d, jnp.zeros((BN, F_PAD - F_IN), jnp.float32)], axis=1)


_pre = pl.pallas_call(
    _pre_body,
    grid=(N // BN,),
    in_specs=[
        pl.BlockSpec((BN, F_IN), lambda i: (i, 0)),
        pl.BlockSpec((BN, 1), lambda i: (i, 0)),
    ],
    out_specs=pl.BlockSpec((BN, F_PAD), lambda i: (i, 0)),
    out_shape=jax.ShapeDtypeStruct((N, F_PAD), jnp.float32),
)


def _mmA_body(t_ref, u_ref, dg_ref, b1_ref, w1_ref, w2_ref, o_ref):
    d = lax.rsqrt(dg_ref[...])
    t = t_ref[0] + t_ref[1] + u_ref[...]
    h = jnp.maximum(
        d * jnp.dot(t[:, :F_IN], w1_ref[...],
                    preferred_element_type=jnp.float32) + b1_ref[...], 0.0)
    o_ref[...] = jnp.dot(h * d, w2_ref[...], preferred_element_type=jnp.float32)


_mmA = pl.pallas_call(
    _mmA_body,
    grid=(N // BN,),
    in_specs=[
        pl.BlockSpec((NCORE, BN, F_PAD), lambda i: (0, i, 0)),
        pl.BlockSpec((BN, F_PAD), lambda i: (i, 0)),
        pl.BlockSpec((BN, 1), lambda i: (i, 0)),
        pl.BlockSpec((1, H), lambda i: (0, 0)),
        pl.BlockSpec((F_IN, H), lambda i: (0, 0)),
        pl.BlockSpec((H, H), lambda i: (0, 0)),
    ],
    out_specs=pl.BlockSpec((BN, H), lambda i: (i, 0)),
    out_shape=jax.ShapeDtypeStruct((N, H), jnp.float32),
)


def _mml_body(s_ref, g_ref, dg_ref, b_ref, w_ref, o_ref):
    d = lax.rsqrt(dg_ref[...])
    h = jnp.maximum(d * (s_ref[...] + g_ref[...]) + b_ref[...], 0.0)
    o_ref[...] = jnp.dot(h * d, w_ref[...], preferred_element_type=jnp.float32)


_mml = pl.pallas_call(
    _mml_body,
    grid=(N // BN,),
    in_specs=[
        pl.BlockSpec((BN, H), lambda i: (i, 0)),
        pl.BlockSpec((BN, H), lambda i: (i, 0)),
        pl.BlockSpec((BN, 1), lambda i: (i, 0)),
        pl.BlockSpec((1, H), lambda i: (0, 0)),
        pl.BlockSpec((H, H), lambda i: (0, 0)),
    ],
    out_specs=pl.BlockSpec((BN, H), lambda i: (i, 0)),
    out_shape=jax.ShapeDtypeStruct((N, H), jnp.float32),
)


def _pool_body(s_ref, g_ref, dg_ref, b_ref, batch_ref, w_ref, bo_ref,
               o_ref, p_acc):
    i = pl.program_id(0)
    d = lax.rsqrt(dg_ref[...])
    h = jnp.maximum(d * (s_ref[...] + g_ref[...]) + b_ref[...], 0.0)
    hh = jnp.concatenate([h, jnp.ones((BN, 1), jnp.float32)], axis=1)
    oh = (batch_ref[...] ==
          lax.broadcasted_iota(jnp.int32, (BN, G), 1)).astype(jnp.float32)
    contrib = lax.dot_general(oh, hh, (((0,), (0,)), ((), ())),
                              preferred_element_type=jnp.float32)

    @pl.when(i == 0)
    def _():
        p_acc[...] = contrib

    @pl.when(i != 0)
    def _():
        p_acc[...] = p_acc[...] + contrib

    @pl.when(i == N // BN - 1)
    def _():
        P = p_acc[...]
        cnt = jnp.maximum(P[:, H:H + 1], 1.0)
        pooled = P[:, :H] / cnt
        logits = jnp.dot(pooled, w_ref[...],
                         preferred_element_type=jnp.float32) + bo_ref[...]
        m = jnp.max(logits, axis=1, keepdims=True)
        e = jnp.exp(logits - m)
        lse = jnp.log(jnp.sum(e, axis=1, keepdims=True)) + m
        o_ref[...] = logits - lse


_pool = pl.pallas_call(
    _pool_body,
    grid=(N // BN,),
    in_specs=[
        pl.BlockSpec((BN, H), lambda i: (i, 0)),
        pl.BlockSpec((BN, H), lambda i: (i, 0)),
        pl.BlockSpec((BN, 1), lambda i: (i, 0)),
        pl.BlockSpec((1, H), lambda i: (0, 0)),
        pl.BlockSpec((BN, 1), lambda i: (i, 0)),
        pl.BlockSpec((H, C_OUT), lambda i: (0, 0)),
        pl.BlockSpec((1, C_OUT), lambda i: (0, 0)),
    ],
    out_specs=pl.BlockSpec((G, C_OUT), lambda i: (0, 0)),
    out_shape=jax.ShapeDtypeStruct((G, C_OUT), jnp.float32),
    scratch_shapes=[pltpu.VMEM((G, H + 1), jnp.float32)],
)


def kernel(x, edge_index, batch, W1, b1, W2, b2, W3, b3, W4, b4, W5, b5,
           Wout, bout):
    src = edge_index[0]
    dst = edge_index[1]
    pad = E_PAD - E
    srcp = jnp.concatenate(
        [src, jnp.arange(pad, dtype=jnp.int32) % N]).reshape(EROWS, CHUNK)
    dstp = jnp.concatenate(
        [dst, N + (jnp.arange(pad, dtype=jnp.int32) % (N_ACC - N))]
    ).reshape(EROWS, CHUNK)

    degp = _deg_kernel(dstp)
    degsum = (1.0 + degp[0, :N] + degp[1, :N]).reshape(N, 1)
    batch2 = batch.reshape(N, 1)

    u1 = _pre(x, degsum)
    T = _scatter8_kernel(u1, srcp, dstp)
    g = _mmA(T, u1, degsum, b1.reshape(1, H), W1, W2)
    for (W, b) in ((W3, b2), (W4, b3), (W5, b4)):
        S = _scatter_kernel(g.reshape(2 * N, HH), srcp, dstp)
        g = _mml(S, g, degsum, b.reshape(1, H), W)
    S = _scatter_kernel(g.reshape(2 * N, HH), srcp, dstp)
    return _pool(S, g, degsum, b5.reshape(1, H), batch2,
                 Wout, bout.reshape(1, C_OUT))


# speedup vs baseline: 2.3034x; 1.0211x over previous
import functools

import jax
import jax.numpy as jnp
from jax import lax
from jax.experimental import pallas as pl
from jax.experimental.pallas import tpu as pltpu
from jax.experimental.pallas import tpu_sc as plsc

N = 50000
E = 800000
F_IN = 8
H = 64
HH = 32
C_OUT = 10
G = 128

NCORE = 2
NSUB = 16
CHUNK = 128
CPB = 8
NBLK = 50
NBLK1 = 25
F_PAD = 16
EPT = CHUNK * CPB * NBLK
E_PAD = EPT * NSUB
EROWS = E_PAD // CHUNK
RPT_E = CPB * NBLK

N_ACC = 50048
RPT = N_ACC // NSUB
ZROWS = 136
WAVE = 5

DEG_PAD = 51200
DPT = DEG_PAD // NSUB

BN = 5000

_MESH = plsc.VectorSubcoreMesh(
    core_axis_name="c", subcore_axis_name="s",
    num_cores=NCORE, num_subcores=NSUB,
)

def _z16():
    return jnp.zeros((16,), jnp.float32)


@functools.partial(
    pl.kernel,
    out_type=jax.ShapeDtypeStruct((NCORE, DEG_PAD), jnp.float32),
    mesh=_MESH,
    compiler_params=pltpu.CompilerParams(use_tc_tiling_on_sc=False),
    scratch_types=[
        pltpu.VMEM_SHARED((DEG_PAD,), jnp.float32),
        pltpu.VMEM((CPB, CHUNK), jnp.int32),
        pltpu.VMEM((CHUNK,), jnp.float32),
        pltpu.VMEM((DPT,), jnp.float32),
    ],
)
def _deg_kernel(dstp, out, acc, dbuf, ones, zbuf):
    c = lax.axis_index("c")
    s = lax.axis_index("s")

    def zfill(i, carry):
        zbuf[pl.ds(i * 16, 16)] = _z16()
        return carry
    lax.fori_loop(0, DPT // 16, zfill, 0)
    pltpu.sync_copy(zbuf, acc.at[pl.ds(s * DPT, DPT)])

    def ofill(i, carry):
        ones[pl.ds(i * 16, 16)] = _z16() + 1.0
        return carry
    lax.fori_loop(0, CHUNK // 16, ofill, 0)
    plsc.subcore_barrier()

    rb0 = (c * NSUB + s) * (NBLK1 * CPB)

    def block(b, carry):
        pltpu.sync_copy(dstp.at[pl.ds(rb0 + b * CPB, CPB)], dbuf)
        for j in range(CPB):
            pltpu.sync_copy(ones, acc.at[dbuf.at[j]], add=True)
        return carry

    lax.fori_loop(0, NBLK1, block, 0)
    plsc.subcore_barrier()
    pltpu.sync_copy(acc.at[pl.ds(s * DPT, DPT)], out.at[c, pl.ds(s * DPT, DPT)])


@functools.partial(
    pl.kernel,
    out_type=jax.ShapeDtypeStruct((NCORE, N_ACC, F_PAD), jnp.float32),
    mesh=_MESH,
    compiler_params=pltpu.CompilerParams(use_tc_tiling_on_sc=False),
    scratch_types=[
        pltpu.VMEM_SHARED((N_ACC, F_PAD), jnp.float32),
        pltpu.VMEM((CPB, CHUNK), jnp.int32),
        pltpu.VMEM((CPB, CHUNK), jnp.int32),
        pltpu.VMEM((CPB, CHUNK), jnp.int32),
        pltpu.VMEM((CPB, CHUNK), jnp.int32),
        pltpu.VMEM((WAVE * CHUNK, F_PAD), jnp.float32),
        pltpu.VMEM((ZROWS, F_PAD), jnp.float32),
        pltpu.SemaphoreType.DMA,
        pltpu.SemaphoreType.DMA,
        pltpu.SemaphoreType.DMA,
        pltpu.SemaphoreType.DMA,
        pltpu.SemaphoreType.DMA,
        pltpu.SemaphoreType.DMA,
        pltpu.SemaphoreType.DMA,
        pltpu.SemaphoreType.DMA,
        pltpu.SemaphoreType.DMA,
        pltpu.SemaphoreType.DMA,
        pltpu.SemaphoreType.DMA,
        pltpu.SemaphoreType.DMA,
    ],
)
def _scatter8_kernel(u, srcp, dstp, out, acc, sbufA, dbufA, sbufB, dbufB,
                     rows, zbuf,
                     g0, g1, g2s, g3, g4, s0, s1, s2, s3, s4, isemA, isemB):
    gsem = [g0, g1, g2s, g3, g4]
    ssem = [s0, s1, s2, s3, s4]
    c = lax.axis_index("c")
    s = lax.axis_index("s")

    def zrow(r, carry):
        zbuf[r, pl.ds(0, 16)] = _z16()
        return carry
    lax.fori_loop(0, ZROWS, zrow, 0)
    row0 = s * RPT
    for jz in range(RPT // ZROWS):
        pltpu.sync_copy(zbuf, acc.at[pl.ds(row0 + jz * ZROWS, ZROWS)])
    plsc.subcore_barrier()

    rb0 = (c * NSUB + s) * (NBLK1 * CPB)

    def fire_idx(b, sb, db, isem):
        rb = rb0 + b * CPB
        pltpu.async_copy(srcp.at[pl.ds(rb, CPB)], sb, isem)
        pltpu.async_copy(dstp.at[pl.ds(rb, CPB)], db, isem)

    def wait_idx(sb, db, isem):
        pltpu.make_async_copy(srcp.at[pl.ds(0, CPB)], sb, isem).wait()
        pltpu.make_async_copy(dstp.at[pl.ds(0, CPB)], db, isem).wait()

    def run_block(sbuf, dbuf):
        gh = {}
        sh = {}
        for j in range(WAVE):
            gh[j] = pltpu.async_copy(
                u.at[sbuf.at[j]], rows.at[pl.ds(j * CHUNK, CHUNK)], gsem[j])
        for j in range(CPB):
            sl = j % WAVE
            gh[j].wait()
            sh[j] = pltpu.async_copy(
                rows.at[pl.ds(sl * CHUNK, CHUNK)], acc.at[dbuf.at[j]],
                ssem[sl], add=True)
            nj = j + WAVE
            if nj < CPB:
                sh[j].wait()
                gh[nj] = pltpu.async_copy(
                    u.at[sbuf.at[nj]], rows.at[pl.ds(sl * CHUNK, CHUNK)],
                    gsem[sl])
        for j in range(CPB - WAVE, CPB):
            sh[j].wait()

    fire_idx(0, sbufA, dbufA, isemA)

    def pair(t, carry):
        b0 = 2 * t
        fire_idx(b0 + 1, sbufB, dbufB, isemB)
        wait_idx(sbufA, dbufA, isemA)
        run_block(sbufA, dbufA)
        fire_idx(b0 + 2, sbufA, dbufA, isemA)
        wait_idx(sbufB, dbufB, isemB)
        run_block(sbufB, dbufB)
        return carry

    lax.fori_loop(0, NBLK1 // 2, pair, 0)
    wait_idx(sbufA, dbufA, isemA)
    run_block(sbufA, dbufA)
    plsc.subcore_barrier()
    pltpu.sync_copy(acc.at[pl.ds(row0, RPT)], out.at[c, pl.ds(row0, RPT)])


@functools.partial(
    pl.kernel,
    out_type=jax.ShapeDtypeStruct((N_ACC, H), jnp.float32),
    mesh=_MESH,
    compiler_params=pltpu.CompilerParams(use_tc_tiling_on_sc=False),
    scratch_types=[
        pltpu.VMEM_SHARED((N_ACC, HH), jnp.float32),
        pltpu.VMEM((CPB, CHUNK), jnp.int32),
        pltpu.VMEM((CPB, CHUNK), jnp.int32),
        pltpu.VMEM((CPB, CHUNK), jnp.int32),
        pltpu.VMEM((CPB, CHUNK), jnp.int32),
        pltpu.VMEM((WAVE * CHUNK, HH), jnp.float32),
        pltpu.VMEM((ZROWS, HH), jnp.float32),
        pltpu.SemaphoreType.DMA,
        pltpu.SemaphoreType.DMA,
        pltpu.SemaphoreType.DMA,
        pltpu.SemaphoreType.DMA,
        pltpu.SemaphoreType.DMA,
        pltpu.SemaphoreType.DMA,
        pltpu.SemaphoreType.DMA,
        pltpu.SemaphoreType.DMA,
        pltpu.SemaphoreType.DMA,
        pltpu.SemaphoreType.DMA,
        pltpu.SemaphoreType.DMA,
        pltpu.SemaphoreType.DMA,
    ],
)
def _scatter_kernel(g2, srcp, dstp, out, acc, sbufA, dbufA, sbufB, dbufB,
                    rows, zbuf,
                    g0, g1, g2s, g3, g4, s0, s1, s2, s3, s4, isemA, isemB):
    gsem = [g0, g1, g2s, g3, g4]
    ssem = [s0, s1, s2, s3, s4]
    c = lax.axis_index("c")
    s = lax.axis_index("s")

    def zrow(r, carry):
        zbuf[r, pl.ds(0, 16)] = _z16()
        zbuf[r, pl.ds(16, 16)] = _z16()
        return carry
    lax.fori_loop(0, ZROWS, zrow, 0)
    row0 = s * RPT
    for jz in range(RPT // ZROWS):
        pltpu.sync_copy(zbuf, acc.at[pl.ds(row0 + jz * ZROWS, ZROWS)])
    plsc.subcore_barrier()

    rb0 = s * RPT_E

    def fire_idx(b, sb, db, isem):
        rb = rb0 + b * CPB
        pltpu.async_copy(srcp.at[pl.ds(rb, CPB)], sb, isem)
        pltpu.async_copy(dstp.at[pl.ds(rb, CPB)], db, isem)

    def wait_idx(sb, db, isem):
        pltpu.make_async_copy(srcp.at[pl.ds(0, CPB)], sb, isem).wait()
        pltpu.make_async_copy(dstp.at[pl.ds(0, CPB)], db, isem).wait()

    def run_block(sbuf, dbuf):
        def xf(i, carry2):
            r = i // 8
            k = (i % 8) * 16
            v = sbuf[r, pl.ds(k, 16)]
            sbuf[r, pl.ds(k, 16)] = v + v + c
            return carry2
        lax.fori_loop(0, CPB * 8, xf, 0)
        gbuf = sbuf

        gh = {}
        sh = {}
        for j in range(WAVE):
            gh[j] = pltpu.async_copy(
                g2.at[gbuf.at[j]], rows.at[pl.ds(j * CHUNK, CHUNK)], gsem[j])
        for j in range(CPB):
            sl = j % WAVE
            gh[j].wait()
            sh[j] = pltpu.async_copy(
                rows.at[pl.ds(sl * CHUNK, CHUNK)], acc.at[dbuf.at[j]],
                ssem[sl], add=True)
            nj = j + WAVE
            if nj < CPB:
                sh[j].wait()
                gh[nj] = pltpu.async_copy(
                    g2.at[gbuf.at[nj]], rows.at[pl.ds(sl * CHUNK, CHUNK)],
                    gsem[sl])
        for j in range(CPB - WAVE, CPB):
            sh[j].wait()

    fire_idx(0, sbufA, dbufA, isemA)

    def pair(t, carry):
        b0 = 2 * t
        fire_idx(b0 + 1, sbufB, dbufB, isemB)
        wait_idx(sbufA, dbufA, isemA)
        run_block(sbufA, dbufA)

        @pl.when(t < NBLK // 2 - 1)
        def _():
            fire_idx(b0 + 2, sbufA, dbufA, isemA)
        wait_idx(sbufB, dbufB, isemB)
        run_block(sbufB, dbufB)
        return carry

    lax.fori_loop(0, NBLK // 2, pair, 0)
    plsc.subcore_barrier()
    pltpu.sync_copy(acc.at[pl.ds(row0, RPT)],
                    out.at[pl.ds(row0, RPT), pl.ds(c * HH, HH)])


def _pre_body(x_ref, dg_ref, o_ref):
    d = lax.rsqrt(dg_ref[...])
    o_ref[...] = jnp.concatenate(
        [x_ref[...] * d, jnp.zeros((BN, F_PAD - F_IN), jnp.float32)], axis=1)


_pre = pl.pallas_call(
    _pre_body,
    grid=(N // BN,),
    in_specs=[
        pl.BlockSpec((BN, F_IN), lambda i: (i, 0)),
        pl.BlockSpec((BN, 1), lambda i: (i, 0)),
    ],
    out_specs=pl.BlockSpec((BN, F_PAD), lambda i: (i, 0)),
    out_shape=jax.ShapeDtypeStruct((N, F_PAD), jnp.float32),
)


def _mmA_body(t_ref, u_ref, dg_ref, b1_ref, w1_ref, w2_ref, o_ref):
    d = lax.rsqrt(dg_ref[...])
    t = t_ref[0] + t_ref[1] + u_ref[...]
    h = jnp.maximum(
        d * jnp.dot(t[:, :F_IN], w1_ref[...],
                    preferred_element_type=jnp.float32) + b1_ref[...], 0.0)
    o_ref[...] = jnp.dot(h * d, w2_ref[...], preferred_element_type=jnp.float32)


_mmA = pl.pallas_call(
    _mmA_body,
    grid=(N // BN,),
    in_specs=[
        pl.BlockSpec((NCORE, BN, F_PAD), lambda i: (0, i, 0)),
        pl.BlockSpec((BN, F_PAD), lambda i: (i, 0)),
        pl.BlockSpec((BN, 1), lambda i: (i, 0)),
        pl.BlockSpec((1, H), lambda i: (0, 0)),
        pl.BlockSpec((F_IN, H), lambda i: (0, 0)),
        pl.BlockSpec((H, H), lambda i: (0, 0)),
    ],
    out_specs=pl.BlockSpec((BN, H), lambda i: (i, 0)),
    out_shape=jax.ShapeDtypeStruct((N, H), jnp.float32),
)


def _mml_body(s_ref, g_ref, dg_ref, b_ref, w_ref, o_ref):
    d = lax.rsqrt(dg_ref[...])
    h = jnp.maximum(d * (s_ref[...] + g_ref[...]) + b_ref[...], 0.0)
    o_ref[...] = jnp.dot(h * d, w_ref[...], preferred_element_type=jnp.float32)


_mml = pl.pallas_call(
    _mml_body,
    grid=(N // BN,),
    in_specs=[
        pl.BlockSpec((BN, H), lambda i: (i, 0)),
        pl.BlockSpec((BN, H), lambda i: (i, 0)),
        pl.BlockSpec((BN, 1), lambda i: (i, 0)),
        pl.BlockSpec((1, H), lambda i: (0, 0)),
        pl.BlockSpec((H, H), lambda i: (0, 0)),
    ],
    out_specs=pl.BlockSpec((BN, H), lambda i: (i, 0)),
    out_shape=jax.ShapeDtypeStruct((N, H), jnp.float32),
)


def _pool_body(s_ref, g_ref, dg_ref, b_ref, batch_ref, w_ref, bo_ref,
               o_ref, p_acc):
    i = pl.program_id(0)
    d = lax.rsqrt(dg_ref[...])
    h = jnp.maximum(d * (s_ref[...] + g_ref[...]) + b_ref[...], 0.0)
    hh = jnp.concatenate([h, jnp.ones((BN, 1), jnp.float32)], axis=1)
    oh = (batch_ref[...] ==
          lax.broadcasted_iota(jnp.int32, (BN, G), 1)).astype(jnp.float32)
    contrib = lax.dot_general(oh, hh, (((0,), (0,)), ((), ())),
                              preferred_element_type=jnp.float32)

    @pl.when(i == 0)
    def _():
        p_acc[...] = contrib

    @pl.when(i != 0)
    def _():
        p_acc[...] = p_acc[...] + contrib

    @pl.when(i == N // BN - 1)
    def _():
        P = p_acc[...]
        cnt = jnp.maximum(P[:, H:H + 1], 1.0)
        pooled = P[:, :H] / cnt
        logits = jnp.dot(pooled, w_ref[...],
                         preferred_element_type=jnp.float32) + bo_ref[...]
        m = jnp.max(logits, axis=1, keepdims=True)
        e = jnp.exp(logits - m)
        lse = jnp.log(jnp.sum(e, axis=1, keepdims=True)) + m
        o_ref[...] = logits - lse


_pool = pl.pallas_call(
    _pool_body,
    grid=(N // BN,),
    in_specs=[
        pl.BlockSpec((BN, H), lambda i: (i, 0)),
        pl.BlockSpec((BN, H), lambda i: (i, 0)),
        pl.BlockSpec((BN, 1), lambda i: (i, 0)),
        pl.BlockSpec((1, H), lambda i: (0, 0)),
        pl.BlockSpec((BN, 1), lambda i: (i, 0)),
        pl.BlockSpec((H, C_OUT), lambda i: (0, 0)),
        pl.BlockSpec((1, C_OUT), lambda i: (0, 0)),
    ],
    out_specs=pl.BlockSpec((G, C_OUT), lambda i: (0, 0)),
    out_shape=jax.ShapeDtypeStruct((G, C_OUT), jnp.float32),
    scratch_shapes=[pltpu.VMEM((G, H + 1), jnp.float32)],
)


def kernel(x, edge_index, batch, W1, b1, W2, b2, W3, b3, W4, b4, W5, b5,
           Wout, bout):
    src = edge_index[0]
    dst = edge_index[1]
    pad = E_PAD - E
    srcp = jnp.concatenate(
        [src, jnp.arange(pad, dtype=jnp.int32) % N]).reshape(EROWS, CHUNK)
    dstp = jnp.concatenate(
        [dst, N + (jnp.arange(pad, dtype=jnp.int32) % (N_ACC - N))]
    ).reshape(EROWS, CHUNK)

    degp = _deg_kernel(dstp)
    degsum = (1.0 + degp[0, :N] + degp[1, :N]).reshape(N, 1)
    batch2 = batch.reshape(N, 1)

    u1 = _pre(x, degsum)
    T = _scatter8_kernel(u1, srcp, dstp)
    g = _mmA(T, u1, degsum, b1.reshape(1, H), W1, W2)
    for (W, b) in ((W3, b2), (W4, b3), (W5, b4)):
        S = _scatter_kernel(g.reshape(2 * N, HH), srcp, dstp)
        g = _mml(S, g, degsum, b.reshape(1, H), W)
    S = _scatter_kernel(g.reshape(2 * N, HH), srcp, dstp)
    return _pool(S, g, degsum, b5.reshape(1, H), batch2,
                 Wout, bout.reshape(1, C_OUT))


# speedup vs baseline: 2.4130x; 1.0476x over previous
import functools

import jax
import jax.numpy as jnp
from jax import lax
from jax.experimental import pallas as pl
from jax.experimental.pallas import tpu as pltpu
from jax.experimental.pallas import tpu_sc as plsc

N = 50000
E = 800000
F_IN = 8
H = 64
HH = 32
C_OUT = 10
G = 128

NCORE = 2
NSUB = 16
CHUNK = 128
CPB = 8
NBLK = 50
NBLK1 = 25
F_PAD = 16
EPT = CHUNK * CPB * NBLK
E_PAD = EPT * NSUB
EROWS = E_PAD // CHUNK
RPT_E = CPB * NBLK

N_ACC = 50048
RPT = N_ACC // NSUB
ZROWS = 136
WAVE = 5
CPB2 = 16
NBLK2 = 25
WAVE2 = 4

DEG_PAD = 51200
DPT = DEG_PAD // NSUB

BN = 5000

_MESH = plsc.VectorSubcoreMesh(
    core_axis_name="c", subcore_axis_name="s",
    num_cores=NCORE, num_subcores=NSUB,
)

def _z16():
    return jnp.zeros((16,), jnp.float32)


@functools.partial(
    pl.kernel,
    out_type=jax.ShapeDtypeStruct((NCORE, DEG_PAD), jnp.float32),
    mesh=_MESH,
    compiler_params=pltpu.CompilerParams(use_tc_tiling_on_sc=False),
    scratch_types=[
        pltpu.VMEM_SHARED((DEG_PAD,), jnp.float32),
        pltpu.VMEM((CPB, CHUNK), jnp.int32),
        pltpu.VMEM((CHUNK,), jnp.float32),
        pltpu.VMEM((DPT,), jnp.float32),
    ],
)
def _deg_kernel(dstp, out, acc, dbuf, ones, zbuf):
    c = lax.axis_index("c")
    s = lax.axis_index("s")

    def zfill(i, carry):
        zbuf[pl.ds(i * 16, 16)] = _z16()
        return carry
    lax.fori_loop(0, DPT // 16, zfill, 0)
    pltpu.sync_copy(zbuf, acc.at[pl.ds(s * DPT, DPT)])

    def ofill(i, carry):
        ones[pl.ds(i * 16, 16)] = _z16() + 1.0
        return carry
    lax.fori_loop(0, CHUNK // 16, ofill, 0)
    plsc.subcore_barrier()

    rb0 = (c * NSUB + s) * (NBLK1 * CPB)

    def block(b, carry):
        pltpu.sync_copy(dstp.at[pl.ds(rb0 + b * CPB, CPB)], dbuf)
        for j in range(CPB):
            pltpu.sync_copy(ones, acc.at[dbuf.at[j]], add=True)
        return carry

    lax.fori_loop(0, NBLK1, block, 0)
    plsc.subcore_barrier()
    pltpu.sync_copy(acc.at[pl.ds(s * DPT, DPT)], out.at[c, pl.ds(s * DPT, DPT)])


@functools.partial(
    pl.kernel,
    out_type=jax.ShapeDtypeStruct((NCORE, N_ACC, F_PAD), jnp.float32),
    mesh=_MESH,
    compiler_params=pltpu.CompilerParams(use_tc_tiling_on_sc=False),
    scratch_types=[
        pltpu.VMEM_SHARED((N_ACC, F_PAD), jnp.float32),
        pltpu.VMEM((CPB, CHUNK), jnp.int32),
        pltpu.VMEM((CPB, CHUNK), jnp.int32),
        pltpu.VMEM((CPB, CHUNK), jnp.int32),
        pltpu.VMEM((CPB, CHUNK), jnp.int32),
        pltpu.VMEM((WAVE * CHUNK, F_PAD), jnp.float32),
        pltpu.VMEM((ZROWS, F_PAD), jnp.float32),
        pltpu.SemaphoreType.DMA,
        pltpu.SemaphoreType.DMA,
        pltpu.SemaphoreType.DMA,
        pltpu.SemaphoreType.DMA,
        pltpu.SemaphoreType.DMA,
        pltpu.SemaphoreType.DMA,
        pltpu.SemaphoreType.DMA,
        pltpu.SemaphoreType.DMA,
        pltpu.SemaphoreType.DMA,
        pltpu.SemaphoreType.DMA,
        pltpu.SemaphoreType.DMA,
        pltpu.SemaphoreType.DMA,
    ],
)
def _scatter8_kernel(u, srcp, dstp, out, acc, sbufA, dbufA, sbufB, dbufB,
                     rows, zbuf,
                     g0, g1, g2s, g3, g4, s0, s1, s2, s3, s4, isemA, isemB):
    gsem = [g0, g1, g2s, g3, g4]
    ssem = [s0, s1, s2, s3, s4]
    c = lax.axis_index("c")
    s = lax.axis_index("s")

    def zrow(r, carry):
        zbuf[r, pl.ds(0, 16)] = _z16()
        return carry
    lax.fori_loop(0, ZROWS, zrow, 0)
    row0 = s * RPT
    for jz in range(RPT // ZROWS):
        pltpu.sync_copy(zbuf, acc.at[pl.ds(row0 + jz * ZROWS, ZROWS)])
    plsc.subcore_barrier()

    rb0 = (c * NSUB + s) * (NBLK1 * CPB)

    def fire_idx(b, sb, db, isem):
        rb = rb0 + b * CPB
        pltpu.async_copy(srcp.at[pl.ds(rb, CPB)], sb, isem)
        pltpu.async_copy(dstp.at[pl.ds(rb, CPB)], db, isem)

    def wait_idx(sb, db, isem):
        pltpu.make_async_copy(srcp.at[pl.ds(0, CPB)], sb, isem).wait()
        pltpu.make_async_copy(dstp.at[pl.ds(0, CPB)], db, isem).wait()

    def run_block(sbuf, dbuf):
        gh = {}
        sh = {}
        for j in range(WAVE):
            gh[j] = pltpu.async_copy(
                u.at[sbuf.at[j]], rows.at[pl.ds(j * CHUNK, CHUNK)], gsem[j])
        for j in range(CPB):
            sl = j % WAVE
            gh[j].wait()
            sh[j] = pltpu.async_copy(
                rows.at[pl.ds(sl * CHUNK, CHUNK)], acc.at[dbuf.at[j]],
                ssem[sl], add=True)
            nj = j + WAVE
            if nj < CPB:
                sh[j].wait()
                gh[nj] = pltpu.async_copy(
                    u.at[sbuf.at[nj]], rows.at[pl.ds(sl * CHUNK, CHUNK)],
                    gsem[sl])
        for j in range(CPB - WAVE, CPB):
            sh[j].wait()

    fire_idx(0, sbufA, dbufA, isemA)

    def pair(t, carry):
        b0 = 2 * t
        fire_idx(b0 + 1, sbufB, dbufB, isemB)
        wait_idx(sbufA, dbufA, isemA)
        run_block(sbufA, dbufA)
        fire_idx(b0 + 2, sbufA, dbufA, isemA)
        wait_idx(sbufB, dbufB, isemB)
        run_block(sbufB, dbufB)
        return carry

    lax.fori_loop(0, NBLK1 // 2, pair, 0)
    wait_idx(sbufA, dbufA, isemA)
    run_block(sbufA, dbufA)
    plsc.subcore_barrier()
    pltpu.sync_copy(acc.at[pl.ds(row0, RPT)], out.at[c, pl.ds(row0, RPT)])


@functools.partial(
    pl.kernel,
    out_type=jax.ShapeDtypeStruct((N_ACC, H), jnp.float32),
    mesh=_MESH,
    compiler_params=pltpu.CompilerParams(use_tc_tiling_on_sc=False),
    scratch_types=[
        pltpu.VMEM_SHARED((N_ACC, HH), jnp.float32),
        pltpu.VMEM((CPB2, CHUNK), jnp.int32),
        pltpu.VMEM((CPB2, CHUNK), jnp.int32),
        pltpu.VMEM((CPB2, CHUNK), jnp.int32),
        pltpu.VMEM((CPB2, CHUNK), jnp.int32),
        pltpu.VMEM((WAVE2 * CHUNK, HH), jnp.float32),
        pltpu.VMEM((ZROWS, HH), jnp.float32),
        pltpu.SemaphoreType.DMA,
        pltpu.SemaphoreType.DMA,
        pltpu.SemaphoreType.DMA,
        pltpu.SemaphoreType.DMA,
        pltpu.SemaphoreType.DMA,
        pltpu.SemaphoreType.DMA,
        pltpu.SemaphoreType.DMA,
        pltpu.SemaphoreType.DMA,
        pltpu.SemaphoreType.DMA,
        pltpu.SemaphoreType.DMA,
    ],
)
def _scatter_kernel(g2, srcp, dstp, out, acc, sbufA, dbufA, sbufB, dbufB,
                    rows, zbuf,
                    g0, g1, g2s, g3, s0, s1, s2, s3, isemA, isemB):
    gsem = [g0, g1, g2s, g3]
    ssem = [s0, s1, s2, s3]
    c = lax.axis_index("c")
    s = lax.axis_index("s")

    def zrow(r, carry):
        zbuf[r, pl.ds(0, 16)] = _z16()
        zbuf[r, pl.ds(16, 16)] = _z16()
        return carry
    lax.fori_loop(0, ZROWS, zrow, 0)
    row0 = s * RPT
    for jz in range(RPT // ZROWS):
        pltpu.sync_copy(zbuf, acc.at[pl.ds(row0 + jz * ZROWS, ZROWS)])
    plsc.subcore_barrier()

    rb0 = s * RPT_E

    def fire_idx(b, sb, db, isem):
        rb = rb0 + b * CPB2
        pltpu.async_copy(srcp.at[pl.ds(rb, CPB2)], sb, isem)
        pltpu.async_copy(dstp.at[pl.ds(rb, CPB2)], db, isem)

    def wait_idx(sb, db, isem):
        pltpu.make_async_copy(srcp.at[pl.ds(0, CPB2)], sb, isem).wait()
        pltpu.make_async_copy(dstp.at[pl.ds(0, CPB2)], db, isem).wait()

    def run_block(sbuf, dbuf):
        def xf(i, carry2):
            r = i // 8
            k = (i % 8) * 16
            v = sbuf[r, pl.ds(k, 16)]
            sbuf[r, pl.ds(k, 16)] = v + v + c
            return carry2
        lax.fori_loop(0, CPB2 * 8, xf, 0)
        gbuf = sbuf

        gh = {}
        sh = {}
        for j in range(WAVE2):
            gh[j] = pltpu.async_copy(
                g2.at[gbuf.at[j]], rows.at[pl.ds(j * CHUNK, CHUNK)], gsem[j])
        for j in range(CPB2):
            sl = j % WAVE2
            gh[j].wait()
            sh[j] = pltpu.async_copy(
                rows.at[pl.ds(sl * CHUNK, CHUNK)], acc.at[dbuf.at[j]],
                ssem[sl], add=True)
            nj = j + WAVE2
            if nj < CPB2:
                sh[j].wait()
                gh[nj] = pltpu.async_copy(
                    g2.at[gbuf.at[nj]], rows.at[pl.ds(sl * CHUNK, CHUNK)],
                    gsem[sl])
        for j in range(CPB2 - WAVE2, CPB2):
            sh[j].wait()

    fire_idx(0, sbufA, dbufA, isemA)

    def pair(t, carry):
        b0 = 2 * t
        fire_idx(b0 + 1, sbufB, dbufB, isemB)
        wait_idx(sbufA, dbufA, isemA)
        run_block(sbufA, dbufA)
        fire_idx(b0 + 2, sbufA, dbufA, isemA)
        wait_idx(sbufB, dbufB, isemB)
        run_block(sbufB, dbufB)
        return carry

    lax.fori_loop(0, NBLK2 // 2, pair, 0)
    wait_idx(sbufA, dbufA, isemA)
    run_block(sbufA, dbufA)
    plsc.subcore_barrier()
    pltpu.sync_copy(acc.at[pl.ds(row0, RPT)],
                    out.at[pl.ds(row0, RPT), pl.ds(c * HH, HH)])


def _pre_body(x_ref, dg_ref, o_ref):
    d = lax.rsqrt(dg_ref[...])
    o_ref[...] = jnp.concatenate(
        [x_ref[...] * d, jnp.zeros((BN, F_PAD - F_IN), jnp.float32)], axis=1)


_pre = pl.pallas_call(
    _pre_body,
    grid=(N // BN,),
    in_specs=[
        pl.BlockSpec((BN, F_IN), lambda i: (i, 0)),
        pl.BlockSpec((BN, 1), lambda i: (i, 0)),
    ],
    out_specs=pl.BlockSpec((BN, F_PAD), lambda i: (i, 0)),
    out_shape=jax.ShapeDtypeStruct((N, F_PAD), jnp.float32),
)


def _mmA_body(t_ref, u_ref, dg_ref, b1_ref, w1_ref, w2_ref, o_ref):
    d = lax.rsqrt(dg_ref[...])
    t = t_ref[0] + t_ref[1] + u_ref[...]
    h = jnp.maximum(
        d * jnp.dot(t[:, :F_IN], w1_ref[...],
                    preferred_element_type=jnp.float32) + b1_ref[...], 0.0)
    o_ref[...] = jnp.dot(h * d, w2_ref[...], preferred_element_type=jnp.float32)


_mmA = pl.pallas_call(
    _mmA_body,
    grid=(N // BN,),
    in_specs=[
        pl.BlockSpec((NCORE, BN, F_PAD), lambda i: (0, i, 0)),
        pl.BlockSpec((BN, F_PAD), lambda i: (i, 0)),
        pl.BlockSpec((BN, 1), lambda i: (i, 0)),
        pl.BlockSpec((1, H), lambda i: (0, 0)),
        pl.BlockSpec((F_IN, H), lambda i: (0, 0)),
        pl.BlockSpec((H, H), lambda i: (0, 0)),
    ],
    out_specs=pl.BlockSpec((BN, H), lambda i: (i, 0)),
    out_shape=jax.ShapeDtypeStruct((N, H), jnp.float32),
)


def _mml_body(s_ref, g_ref, dg_ref, b_ref, w_ref, o_ref):
    d = lax.rsqrt(dg_ref[...])
    h = jnp.maximum(d * (s_ref[...] + g_ref[...]) + b_ref[...], 0.0)
    o_ref[...] = jnp.dot(h * d, w_ref[...], preferred_element_type=jnp.float32)


_mml = pl.pallas_call(
    _mml_body,
    grid=(N // BN,),
    in_specs=[
        pl.BlockSpec((BN, H), lambda i: (i, 0)),
        pl.BlockSpec((BN, H), lambda i: (i, 0)),
        pl.BlockSpec((BN, 1), lambda i: (i, 0)),
        pl.BlockSpec((1, H), lambda i: (0, 0)),
        pl.BlockSpec((H, H), lambda i: (0, 0)),
    ],
    out_specs=pl.BlockSpec((BN, H), lambda i: (i, 0)),
    out_shape=jax.ShapeDtypeStruct((N, H), jnp.float32),
)


def _pool_body(s_ref, g_ref, dg_ref, b_ref, batch_ref, w_ref, bo_ref,
               o_ref, p_acc):
    i = pl.program_id(0)
    d = lax.rsqrt(dg_ref[...])
    h = jnp.maximum(d * (s_ref[...] + g_ref[...]) + b_ref[...], 0.0)
    hh = jnp.concatenate([h, jnp.ones((BN, 1), jnp.float32)], axis=1)
    oh = (batch_ref[...] ==
          lax.broadcasted_iota(jnp.int32, (BN, G), 1)).astype(jnp.float32)
    contrib = lax.dot_general(oh, hh, (((0,), (0,)), ((), ())),
                              preferred_element_type=jnp.float32)

    @pl.when(i == 0)
    def _():
        p_acc[...] = contrib

    @pl.when(i != 0)
    def _():
        p_acc[...] = p_acc[...] + contrib

    @pl.when(i == N // BN - 1)
    def _():
        P = p_acc[...]
        cnt = jnp.maximum(P[:, H:H + 1], 1.0)
        pooled = P[:, :H] / cnt
        logits = jnp.dot(pooled, w_ref[...],
                         preferred_element_type=jnp.float32) + bo_ref[...]
        m = jnp.max(logits, axis=1, keepdims=True)
        e = jnp.exp(logits - m)
        lse = jnp.log(jnp.sum(e, axis=1, keepdims=True)) + m
        o_ref[...] = logits - lse


_pool = pl.pallas_call(
    _pool_body,
    grid=(N // BN,),
    in_specs=[
        pl.BlockSpec((BN, H), lambda i: (i, 0)),
        pl.BlockSpec((BN, H), lambda i: (i, 0)),
        pl.BlockSpec((BN, 1), lambda i: (i, 0)),
        pl.BlockSpec((1, H), lambda i: (0, 0)),
        pl.BlockSpec((BN, 1), lambda i: (i, 0)),
        pl.BlockSpec((H, C_OUT), lambda i: (0, 0)),
        pl.BlockSpec((1, C_OUT), lambda i: (0, 0)),
    ],
    out_specs=pl.BlockSpec((G, C_OUT), lambda i: (0, 0)),
    out_shape=jax.ShapeDtypeStruct((G, C_OUT), jnp.float32),
    scratch_shapes=[pltpu.VMEM((G, H + 1), jnp.float32)],
)


def kernel(x, edge_index, batch, W1, b1, W2, b2, W3, b3, W4, b4, W5, b5,
           Wout, bout):
    src = edge_index[0]
    dst = edge_index[1]
    pad = E_PAD - E
    srcp = jnp.concatenate(
        [src, jnp.arange(pad, dtype=jnp.int32) % N]).reshape(EROWS, CHUNK)
    dstp = jnp.concatenate(
        [dst, N + (jnp.arange(pad, dtype=jnp.int32) % (N_ACC - N))]
    ).reshape(EROWS, CHUNK)

    degp = _deg_kernel(dstp)
    degsum = (1.0 + degp[0, :N] + degp[1, :N]).reshape(N, 1)
    batch2 = batch.reshape(N, 1)

    u1 = _pre(x, degsum)
    T = _scatter8_kernel(u1, srcp, dstp)
    g = _mmA(T, u1, degsum, b1.reshape(1, H), W1, W2)
    for (W, b) in ((W3, b2), (W4, b3), (W5, b4)):
        S = _scatter_kernel(g.reshape(2 * N, HH), srcp, dstp)
        g = _mml(S, g, degsum, b.reshape(1, H), W)
    S = _scatter_kernel(g.reshape(2 * N, HH), srcp, dstp)
    return _pool(S, g, degsum, b5.reshape(1, H), batch2,
                 Wout, bout.reshape(1, C_OUT))
